# Initial kernel scaffold; baseline (speedup 1.0000x reference)
import functools

import jax
import jax.numpy as jnp
from jax import lax
from jax.experimental import pallas as pl
from jax.experimental.pallas import tpu as pltpu
from jax.experimental.pallas import tpu_sc as plsc

H = 64
DN = 128
DE = 16
RO = 1024
PH = 512
LAT = 128
NC = 100
B = 256
N = 8192
E = 16384
NG = 3
NT = NG * N
ET = NG * E
BT = NG * B

NW = 32
EPW = ET // NW
NCH = EPW // 128
RPS = NT // 16

NB = 1024
EB = 512
RB = 512

_F32 = jnp.float32


def _dot(a, b):
    return jnp.dot(a, b, preferred_element_type=_F32)


_MESH = plsc.VectorSubcoreMesh(core_axis_name="c", subcore_axis_name="s")


@functools.partial(
    pl.kernel,
    out_type=jax.ShapeDtypeStruct((ET, H), _F32),
    mesh=_MESH,
    scratch_types=[
        pltpu.VMEM((NCH, 128), jnp.int32),
        pltpu.VMEM((EPW, H), _F32),
        pltpu.SemaphoreType.DMA,
    ],
)
def _sc_gather(h_hbm, idx_hbm, out_hbm, idx_v, rows_v, sem):
    wid = lax.axis_index("s") * 2 + lax.axis_index("c")
    pltpu.sync_copy(idx_hbm.at[wid], idx_v)
    cps = []
    for j in range(NCH):
        cps.append(
            pltpu.async_copy(h_hbm.at[idx_v.at[j]], rows_v.at[pl.ds(j * 128, 128)], sem)
        )
    for cp in cps:
        cp.wait()
    pltpu.sync_copy(rows_v, out_hbm.at[pl.ds(wid * EPW, EPW)])


@functools.partial(
    pl.kernel,
    out_type=jax.ShapeDtypeStruct((2, NT, H), _F32),
    mesh=_MESH,
    scratch_types=[
        pltpu.VMEM((NCH, 128), jnp.int32),
        pltpu.VMEM((EPW, H), _F32),
        pltpu.VMEM((128, H), _F32),
        pltpu.VMEM_SHARED((NT, H), _F32),
    ],
)
def _sc_scatter(m_hbm, idx_hbm, out_hbm, idx_v, rows_v, zbuf, acc):
    c = lax.axis_index("c")
    s = lax.axis_index("s")
    wid = s * 2 + c

    def _zrow(i, carry):
        for k in range(H // 16):
            zbuf[i, pl.ds(k * 16, 16)] = jnp.zeros((16,), _F32)
        return carry

    lax.fori_loop(0, 128, _zrow, 0)
    for j in range(RPS // 128):
        pltpu.sync_copy(zbuf, acc.at[pl.ds(s * RPS + j * 128, 128)])
    plsc.subcore_barrier()

    pltpu.sync_copy(idx_hbm.at[wid], idx_v)
    pltpu.sync_copy(m_hbm.at[pl.ds(wid * EPW, EPW)], rows_v)
    for j in range(NCH):
        pltpu.sync_copy(rows_v.at[pl.ds(j * 128, 128)], acc.at[idx_v.at[j]], add=True)
    plsc.subcore_barrier()

    pltpu.sync_copy(acc.at[pl.ds(s * RPS, RPS)], out_hbm.at[c, pl.ds(s * RPS, RPS)])


def _proj_body(x_ref, w_ref, b_ref, o_ref):
    o_ref[...] = jnp.maximum(_dot(x_ref[...], w_ref[...]) + b_ref[...], 0.0)


def _proj(x, w, b):
    return pl.pallas_call(
        _proj_body,
        grid=(NT // NB,),
        in_specs=[
            pl.BlockSpec((NB, DN), lambda i: (i, 0)),
            pl.BlockSpec((DN, H), lambda i: (0, 0)),
            pl.BlockSpec((1, H), lambda i: (0, 0)),
        ],
        out_specs=pl.BlockSpec((NB, H), lambda i: (i, 0)),
        out_shape=jax.ShapeDtypeStruct((NT, H), _F32),
    )(x, w, b)


def _msg_body(hs_ref, ev_ref, w2_ref, bm_ref, o_ref):
    hs = hs_ref[...]
    ev = ev_ref[...]
    k = (ev[:, :, None] * hs[:, None, :]).reshape(EB, DE * H)
    o_ref[...] = _dot(k, w2_ref[...]) + _dot(hs, bm_ref[...])


def _msg(hs, ev, w2, bmat):
    return pl.pallas_call(
        _msg_body,
        grid=(ET // EB,),
        in_specs=[
            pl.BlockSpec((EB, H), lambda i: (i, 0)),
            pl.BlockSpec((EB, DE), lambda i: (i, 0)),
            pl.BlockSpec((DE * H, H), lambda i: (0, 0)),
            pl.BlockSpec((H, H), lambda i: (0, 0)),
        ],
        out_specs=pl.BlockSpec((EB, H), lambda i: (i, 0)),
        out_shape=jax.ShapeDtypeStruct((ET, H), _F32),
    )(hs, ev, w2, bmat)


def _gru_body(a0_ref, a1_ref, h_ref, cb_ref, wir_ref, wiz_ref, win_ref,
              whr_ref, whz_ref, whn_ref, bir_ref, biz_ref, bin_ref,
              bhr_ref, bhz_ref, bhn_ref, o_ref):
    a = jnp.maximum(a0_ref[...] + a1_ref[...] + cb_ref[...], 0.0)
    h = h_ref[...]
    r = jax.nn.sigmoid(_dot(a, wir_ref[...]) + bir_ref[...]
                       + _dot(h, whr_ref[...]) + bhr_ref[...])
    z = jax.nn.sigmoid(_dot(a, wiz_ref[...]) + biz_ref[...]
                       + _dot(h, whz_ref[...]) + bhz_ref[...])
    n = jnp.tanh(_dot(a, win_ref[...]) + bin_ref[...]
                 + r * (_dot(h, whn_ref[...]) + bhn_ref[...]))
    o_ref[...] = (1.0 - z) * n + z * h


def _gru(a0, a1, h, cb, ws, bs):
    mat = pl.BlockSpec((H, H), lambda i: (0, 0))
    vec = pl.BlockSpec((1, H), lambda i: (0, 0))
    big = pl.BlockSpec((NB, H), lambda i: (i, 0))
    return pl.pallas_call(
        _gru_body,
        grid=(NT // NB,),
        in_specs=[big, big, big, vec] + [mat] * 6 + [vec] * 6,
        out_specs=big,
        out_shape=jax.ShapeDtypeStruct((NT, H), _F32),
    )(a0, a1, h, cb, *ws, *bs)


def _r1_body(feat_ref, gid_ref, q_ref, es_ref, emax_ref):
    i = pl.program_id(0)
    feat = feat_ref[...]
    gid = gid_ref[...]
    segs = lax.broadcasted_iota(_F32, (RB, BT), 1)
    mask = gid == segs
    qg = _dot(mask.astype(_F32), q_ref[...])
    es = jnp.sum(feat * qg, axis=1, keepdims=True)
    es_ref[...] = es

    @pl.when(i == 0)
    def _():
        emax_ref[...] = jnp.full((1, BT), -1e30, _F32)

    part = jnp.max(jnp.where(mask, es, -1e30), axis=0, keepdims=True)
    emax_ref[...] = jnp.maximum(emax_ref[...], part)


def _r1(feat, gidc, q):
    return pl.pallas_call(
        _r1_body,
        grid=(NT // RB,),
        in_specs=[
            pl.BlockSpec((RB, 2 * H), lambda i: (i, 0)),
            pl.BlockSpec((RB, 1), lambda i: (i, 0)),
            pl.BlockSpec((BT, 2 * H), lambda i: (0, 0)),
        ],
        out_specs=[
            pl.BlockSpec((RB, 1), lambda i: (i, 0)),
            pl.BlockSpec((1, BT), lambda i: (0, 0)),
        ],
        out_shape=[
            jax.ShapeDtypeStruct((NT, 1), _F32),
            jax.ShapeDtypeStruct((1, BT), _F32),
        ],
    )(feat, gidc, q)


def _r2_body(feat_ref, gid_ref, es_ref, emax_ref, den_ref, run_ref):
    i = pl.program_id(0)
    feat = feat_ref[...]
    gid = gid_ref[...]
    segs = lax.broadcasted_iota(_F32, (RB, BT), 1)
    mask = gid == segs
    emaxg = jnp.max(jnp.where(mask, emax_ref[...], -1e30), axis=1, keepdims=True)
    ex = jnp.exp(es_ref[...] - emaxg)
    exw = jnp.where(mask, ex, 0.0)

    @pl.when(i == 0)
    def _():
        den_ref[...] = jnp.zeros((BT, 1), _F32)
        run_ref[...] = jnp.zeros((BT, 2 * H), _F32)

    dn = (((0,), (0,)), ((), ()))
    den_ref[...] += lax.dot_general(exw, jnp.ones((RB, 1), _F32), dn,
                                    preferred_element_type=_F32)
    run_ref[...] += lax.dot_general(exw, feat, dn, preferred_element_type=_F32)


def _r2(feat, gidc, es, emax):
    return pl.pallas_call(
        _r2_body,
        grid=(NT // RB,),
        in_specs=[
            pl.BlockSpec((RB, 2 * H), lambda i: (i, 0)),
            pl.BlockSpec((RB, 1), lambda i: (i, 0)),
            pl.BlockSpec((RB, 1), lambda i: (i, 0)),
            pl.BlockSpec((1, BT), lambda i: (0, 0)),
        ],
        out_specs=[
            pl.BlockSpec((BT, 1), lambda i: (0, 0)),
            pl.BlockSpec((BT, 2 * H), lambda i: (0, 0)),
        ],
        out_shape=[
            jax.ShapeDtypeStruct((BT, 1), _F32),
            jax.ShapeDtypeStruct((BT, 2 * H), _F32),
        ],
    )(feat, gidc, es, emax)


def _lstm_body(q_ref, run_ref, den_ref, hl_ref, cl_ref, wq_ref, wr_ref,
               wh_ref, bi_ref, bh_ref, ho_ref, co_ref):
    r = run_ref[...] / jnp.maximum(den_ref[...], 1e-30)
    g = (_dot(q_ref[...], wq_ref[...]) + _dot(r, wr_ref[...])
         + _dot(hl_ref[...], wh_ref[...]) + bi_ref[...] + bh_ref[...])
    gi = jax.nn.sigmoid(g[:, 0 * LAT:1 * LAT])
    gf = jax.nn.sigmoid(g[:, 1 * LAT:2 * LAT])
    gg = jnp.tanh(g[:, 2 * LAT:3 * LAT])
    go = jax.nn.sigmoid(g[:, 3 * LAT:4 * LAT])
    c2 = gf * cl_ref[...] + gi * gg
    ho_ref[...] = go * jnp.tanh(c2)
    co_ref[...] = c2


def _lstm(q, run, den, hl, cl, wq, wr, wh, bi, bh):
    return pl.pallas_call(
        _lstm_body,
        out_shape=[
            jax.ShapeDtypeStruct((BT, 2 * H), _F32),
            jax.ShapeDtypeStruct((BT, 2 * H), _F32),
        ],
    )(q, run, den, hl, cl, wq, wr, wh, bi, bh)


def _sp_body(q_ref, run_ref, den_ref, wq_ref, wr_ref, b_ref, a_ref, o_ref):
    r = run_ref[...] / jnp.maximum(den_ref[...], 1e-30)
    g = _dot(q_ref[...], wq_ref[...]) + _dot(r, wr_ref[...]) + b_ref[...]
    a = a_ref[0, 0]
    o_ref[...] = jnp.where(g >= 0, g, a * g)


def _sp(q, run, den, wq, wr, b, a):
    return pl.pallas_call(
        _sp_body,
        out_shape=jax.ShapeDtypeStruct((BT, RO), _F32),
    )(q, run, den, wq, wr, b, a)


def _lin_body(x_ref, w_ref, b_ref, a_ref, o_ref):
    y = _dot(x_ref[...], w_ref[...]) + b_ref[...]
    a = a_ref[0, 0]
    o_ref[...] = jnp.where(y >= 0, y, a * y)


def _linear_prelu(x, w, b, a):
    return pl.pallas_call(
        _lin_body,
        out_shape=jax.ShapeDtypeStruct((x.shape[0], w.shape[1]), _F32),
    )(x, w, b, a)


def _enc3_body(x_ref, w_ref, b_ref, eps_ref, mu_ref, lv_ref, lat_ref):
    y = _dot(x_ref[...], w_ref[...]) + b_ref[...]
    mu = jnp.clip(y[:, :LAT], -10.0, 10.0)
    lv = jnp.clip(y[:, LAT:], -10.0, 10.0)
    mu_ref[...] = mu
    lv_ref[...] = lv
    lat_ref[...] = mu + eps_ref[...] * jnp.exp(0.5 * lv)


def _enc3(x, w, b, eps):
    return pl.pallas_call(
        _enc3_body,
        out_shape=[
            jax.ShapeDtypeStruct((B, LAT), _F32),
            jax.ShapeDtypeStruct((B, LAT), _F32),
            jax.ShapeDtypeStruct((B, LAT), _F32),
        ],
    )(x, w, b, eps)


def _dec3_body(x_ref, w_ref, b_ref, o_ref):
    o_ref[...] = jnp.clip(_dot(x_ref[...], w_ref[...]) + b_ref[...], -10.0, 10.0)


def _dec3(x, w, b):
    return pl.pallas_call(
        _dec3_body,
        out_shape=jax.ShapeDtypeStruct((B, 128), _F32),
    )(x, w, b)


def kernel(r1_x, r1_e, r1_src, r1_dst, r1_gid, r2_x, r2_e, r2_src, r2_dst,
           r2_gid, pm_x, pm_e, pm_src, pm_dst, pm_gid, labels,
           pos_neg_sample, params):
    p = params
    f32 = _F32

    x = jnp.concatenate([r1_x, r2_x, pm_x], axis=0)
    ev = jnp.concatenate([r1_e, r2_e, pm_e], axis=0)
    src = jnp.concatenate(
        [r1_src, r2_src + N, pm_src + 2 * N]).astype(jnp.int32).reshape(NW, NCH, 128)
    dst = jnp.concatenate(
        [r1_dst, r2_dst + N, pm_dst + 2 * N]).astype(jnp.int32).reshape(NW, NCH, 128)
    gidc = jnp.concatenate(
        [r1_gid, r2_gid + B, pm_gid + 2 * B]).astype(f32).reshape(NT, 1)

    w2 = p['edge_W'].reshape(DE * H, H)
    bmat = p['edge_b'].reshape(H, H)
    cb = p['conv_b'].reshape(1, H)
    gws = ([p['gru_Wih'][k * H:(k + 1) * H].T for k in range(3)]
           + [p['gru_Whh'][k * H:(k + 1) * H].T for k in range(3)])
    gbs = ([p['gru_bih'][k * H:(k + 1) * H].reshape(1, H) for k in range(3)]
           + [p['gru_bhh'][k * H:(k + 1) * H].reshape(1, H) for k in range(3)])

    x0 = _proj(x, p['proj_W'], p['proj_b'].reshape(1, H))
    h = x0
    for _ in range(3):
        hs = _sc_gather(h, src)
        m = _msg(hs, ev, w2, bmat)
        agg2 = _sc_scatter(m, dst)
        h = _gru(agg2[0], agg2[1], h, cb, gws, gbs)

    feat = jnp.concatenate([x0, h], axis=1)

    lwq = p['lstm_Wih'][:, :2 * H].T
    lwr = p['lstm_Wih'][:, 2 * H:].T
    lwh = p['lstm_Whh'].T
    lbi = p['lstm_bih'].reshape(1, 8 * H)
    lbh = p['lstm_bhh'].reshape(1, 8 * H)
    q = jnp.zeros((BT, 2 * H), f32)
    run = jnp.zeros((BT, 2 * H), f32)
    den = jnp.ones((BT, 1), f32)
    hl = jnp.zeros((BT, 2 * H), f32)
    cl = jnp.zeros((BT, 2 * H), f32)
    for _ in range(3):
        hl, cl = _lstm(q, run, den, hl, cl, lwq, lwr, lwh, lbi, lbh)
        q = hl
        es, emax = _r1(feat, gidc, q)
        den, run = _r2(feat, gidc, es, emax)

    g = _sp(q, run, den, p['sp_W'][:2 * H], p['sp_W'][2 * H:],
            p['sp_b'].reshape(1, RO), p['sp_a'].reshape(1, 1))
    ge = g.reshape(NG, B, RO).transpose(1, 0, 2).reshape(B, NG * RO)

    pn = jnp.full((B, 1), pos_neg_sample, f32)
    ei = NG * RO + NC + 1
    eip = 3328
    xin = jnp.concatenate(
        [labels, ge, pn, jnp.zeros((B, eip - ei), f32)], axis=1)
    ew0 = jnp.concatenate(
        [p['enc_W0'], jnp.zeros((eip - ei, PH), f32)], axis=0)
    z = _linear_prelu(xin, ew0, p['enc_b0'].reshape(1, PH), p['enc_a0'].reshape(1, 1))
    z = _linear_prelu(z, p['enc_W1'], p['enc_b1'].reshape(1, PH), p['enc_a1'].reshape(1, 1))
    z = _linear_prelu(z, p['enc_W2'], p['enc_b2'].reshape(1, PH), p['enc_a2'].reshape(1, 1))
    eps = jax.random.normal(jax.random.key(42), (B, LAT), f32)
    mu, log_var, latent = _enc3(z, p['enc_W3'], p['enc_b3'].reshape(1, 2 * LAT), eps)

    di = NG * RO + LAT + 1
    dip = 3328
    yin = jnp.concatenate(
        [latent, ge, pn, jnp.zeros((B, dip - di), f32)], axis=1)
    dw0 = jnp.concatenate(
        [p['dec_W0'], jnp.zeros((dip - di, PH), f32)], axis=0)
    y = _linear_prelu(yin, dw0, p['dec_b0'].reshape(1, PH), p['dec_a0'].reshape(1, 1))
    y = _linear_prelu(y, p['dec_W1'], p['dec_b1'].reshape(1, PH), p['dec_a1'].reshape(1, 1))
    y = _linear_prelu(y, p['dec_W2'], p['dec_b2'].reshape(1, PH), p['dec_a2'].reshape(1, 1))
    dw3 = jnp.pad(p['dec_W3'], ((0, 0), (0, 128 - NC)))
    db3 = jnp.pad(p['dec_b3'], (0, 128 - NC)).reshape(1, 128)
    y = _dec3(y, dw3, db3)[:, :NC]
    return (y, mu, log_var)


# speedup vs baseline: 2.6274x; 2.6274x over previous
import functools

import jax
import jax.numpy as jnp
from jax import lax
from jax.experimental import pallas as pl
from jax.experimental.pallas import tpu as pltpu
from jax.experimental.pallas import tpu_sc as plsc

H = 64
DN = 128
DE = 16
RO = 1024
PH = 512
LAT = 128
NC = 100
B = 256
N = 8192
E = 16384
NG = 3
NT = NG * N
ET = NG * E
BT = NG * B

NW = 32
EPW = ET // NW
NCH = EPW // 128
RPS = NT // 16
HH = H // 4
EPS = ET // 16
NCS = EPS // 128

NB = 1024
EB = 512
RB = 512

_F32 = jnp.float32


def _dot(a, b):
    return jnp.dot(a, b, preferred_element_type=_F32)


def _sc_gather_body(h_hbm, idx_hbm, out_hbm, idx_v, rows_v, sem):
    wid = lax.axis_index("s") * 2 + lax.axis_index("c")
    pltpu.sync_copy(idx_hbm.at[wid], idx_v)
    cps = []
    for j in range(NCH):
        cps.append(
            pltpu.async_copy(h_hbm.at[idx_v.at[j]], rows_v.at[pl.ds(j * 128, 128)], sem)
        )
    for cp in cps:
        cp.wait()
    pltpu.sync_copy(rows_v, out_hbm.at[pl.ds(wid * EPW, EPW)])


def _sc_scatter_body(p, m_hbm, idx_hbm, out_hbm, idx_v, rows_v, zbuf, acc):
    c = lax.axis_index("c")
    s = lax.axis_index("s")

    def _zrow(i, carry):
        zbuf[i, pl.ds(0, 16)] = jnp.zeros((16,), _F32)
        return carry

    lax.fori_loop(0, 128, _zrow, 0)
    for j in range(RPS // 128):
        pltpu.sync_copy(zbuf, acc.at[pl.ds(s * RPS + j * 128, 128)])
    plsc.subcore_barrier()

    pltpu.sync_copy(idx_hbm.at[s], idx_v)
    pltpu.sync_copy(
        m_hbm.at[pl.ds(s * EPS, EPS), pl.ds(2 * p * HH + c * HH, HH)], rows_v)
    for j in range(NCS):
        pltpu.sync_copy(rows_v.at[pl.ds(j * 128, 128)], acc.at[idx_v.at[j]], add=True)
    plsc.subcore_barrier()

    pltpu.sync_copy(acc.at[pl.ds(s * RPS, RPS)], out_hbm.at[c, pl.ds(s * RPS, RPS)])


@functools.lru_cache(maxsize=None)
def _sc_kernels():
    mesh = plsc.VectorSubcoreMesh(core_axis_name="c", subcore_axis_name="s")
    gather = pl.kernel(
        _sc_gather_body,
        out_type=jax.ShapeDtypeStruct((ET, H), _F32),
        mesh=mesh,
        compiler_params=pltpu.CompilerParams(use_tc_tiling_on_sc=False),
        scratch_types=[
            pltpu.VMEM((NCH, 128), jnp.int32),
            pltpu.VMEM((EPW, H), _F32),
            pltpu.SemaphoreType.DMA,
        ],
    )
    scatters = []
    for p in range(2):
        scatters.append(pl.kernel(
            functools.partial(_sc_scatter_body, p),
            out_type=jax.ShapeDtypeStruct((2, NT, HH), _F32),
            mesh=mesh,
            compiler_params=pltpu.CompilerParams(use_tc_tiling_on_sc=False),
            scratch_types=[
                pltpu.VMEM((NCS, 128), jnp.int32),
                pltpu.VMEM((EPS, HH), _F32),
                pltpu.VMEM((128, HH), _F32),
                pltpu.VMEM_SHARED((NT, HH), _F32),
            ],
        ))
    return gather, scatters


def _sc_gather(h, src):
    return _sc_kernels()[0](h, src)


def _sc_scatter(m, dst):
    ks = _sc_kernels()[1]
    o0 = ks[0](m, dst)
    o1 = ks[1](m, dst)
    return jnp.concatenate([o0[0], o0[1], o1[0], o1[1]], axis=1)


def _proj_body(x_ref, w_ref, b_ref, o_ref):
    o_ref[...] = jnp.maximum(_dot(x_ref[...], w_ref[...]) + b_ref[...], 0.0)


def _proj(x, w, b):
    return pl.pallas_call(
        _proj_body,
        grid=(NT // NB,),
        in_specs=[
            pl.BlockSpec((NB, DN), lambda i: (i, 0)),
            pl.BlockSpec((DN, H), lambda i: (0, 0)),
            pl.BlockSpec((1, H), lambda i: (0, 0)),
        ],
        out_specs=pl.BlockSpec((NB, H), lambda i: (i, 0)),
        out_shape=jax.ShapeDtypeStruct((NT, H), _F32),
    )(x, w, b)


def _msg_body(hs_ref, ev_ref, w2_ref, bm_ref, o_ref):
    hs = hs_ref[...]
    ev = ev_ref[...]
    k = (ev[:, :, None] * hs[:, None, :]).reshape(EB, DE * H)
    o_ref[...] = _dot(k, w2_ref[...]) + _dot(hs, bm_ref[...])


def _msg(hs, ev, w2, bmat):
    return pl.pallas_call(
        _msg_body,
        grid=(ET // EB,),
        in_specs=[
            pl.BlockSpec((EB, H), lambda i: (i, 0)),
            pl.BlockSpec((EB, DE), lambda i: (i, 0)),
            pl.BlockSpec((DE * H, H), lambda i: (0, 0)),
            pl.BlockSpec((H, H), lambda i: (0, 0)),
        ],
        out_specs=pl.BlockSpec((EB, H), lambda i: (i, 0)),
        out_shape=jax.ShapeDtypeStruct((ET, H), _F32),
    )(hs, ev, w2, bmat)


def _gru_body(ag_ref, h_ref, cb_ref, wir_ref, wiz_ref, win_ref,
              whr_ref, whz_ref, whn_ref, bir_ref, biz_ref, bin_ref,
              bhr_ref, bhz_ref, bhn_ref, o_ref):
    a = jnp.maximum(ag_ref[...] + cb_ref[...], 0.0)
    h = h_ref[...]
    r = jax.nn.sigmoid(_dot(a, wir_ref[...]) + bir_ref[...]
                       + _dot(h, whr_ref[...]) + bhr_ref[...])
    z = jax.nn.sigmoid(_dot(a, wiz_ref[...]) + biz_ref[...]
                       + _dot(h, whz_ref[...]) + bhz_ref[...])
    n = jnp.tanh(_dot(a, win_ref[...]) + bin_ref[...]
                 + r * (_dot(h, whn_ref[...]) + bhn_ref[...]))
    o_ref[...] = (1.0 - z) * n + z * h


def _gru(ag, h, cb, ws, bs):
    mat = pl.BlockSpec((H, H), lambda i: (0, 0))
    vec = pl.BlockSpec((1, H), lambda i: (0, 0))
    big = pl.BlockSpec((NB, H), lambda i: (i, 0))
    return pl.pallas_call(
        _gru_body,
        grid=(NT // NB,),
        in_specs=[big, big, vec] + [mat] * 6 + [vec] * 6,
        out_specs=big,
        out_shape=jax.ShapeDtypeStruct((NT, H), _F32),
    )(ag, h, cb, *ws, *bs)


def _r1_body(feat_ref, gid_ref, q_ref, es_ref, emax_ref):
    i = pl.program_id(0)
    feat = feat_ref[...]
    gid = gid_ref[...]
    segs = lax.broadcasted_iota(jnp.int32, (RB, BT), 1).astype(_F32)
    mask = gid == segs
    qg = _dot(mask.astype(_F32), q_ref[...])
    es = jnp.sum(feat * qg, axis=1, keepdims=True)
    es_ref[...] = es

    @pl.when(i == 0)
    def _():
        emax_ref[...] = jnp.full((1, BT), -1e30, _F32)

    part = jnp.max(jnp.where(mask, es, -1e30), axis=0, keepdims=True)
    emax_ref[...] = jnp.maximum(emax_ref[...], part)


def _r1(feat, gidc, q):
    return pl.pallas_call(
        _r1_body,
        grid=(NT // RB,),
        in_specs=[
            pl.BlockSpec((RB, 2 * H), lambda i: (i, 0)),
            pl.BlockSpec((RB, 1), lambda i: (i, 0)),
            pl.BlockSpec((BT, 2 * H), lambda i: (0, 0)),
        ],
        out_specs=[
            pl.BlockSpec((RB, 1), lambda i: (i, 0)),
            pl.BlockSpec((1, BT), lambda i: (0, 0)),
        ],
        out_shape=[
            jax.ShapeDtypeStruct((NT, 1), _F32),
            jax.ShapeDtypeStruct((1, BT), _F32),
        ],
    )(feat, gidc, q)


def _r2_body(feat_ref, gid_ref, es_ref, emax_ref, den_ref, run_ref):
    i = pl.program_id(0)
    feat = feat_ref[...]
    gid = gid_ref[...]
    segs = lax.broadcasted_iota(jnp.int32, (RB, BT), 1).astype(_F32)
    mask = gid == segs
    emaxg = jnp.max(jnp.where(mask, emax_ref[...], -1e30), axis=1, keepdims=True)
    ex = jnp.exp(es_ref[...] - emaxg)
    exw = jnp.where(mask, ex, 0.0)

    @pl.when(i == 0)
    def _():
        den_ref[...] = jnp.zeros((BT, 1), _F32)
        run_ref[...] = jnp.zeros((BT, 2 * H), _F32)

    dn = (((0,), (0,)), ((), ()))
    den_ref[...] += lax.dot_general(exw, jnp.ones((RB, 1), _F32), dn,
                                    preferred_element_type=_F32)
    run_ref[...] += lax.dot_general(exw, feat, dn, preferred_element_type=_F32)


def _r2(feat, gidc, es, emax):
    return pl.pallas_call(
        _r2_body,
        grid=(NT // RB,),
        in_specs=[
            pl.BlockSpec((RB, 2 * H), lambda i: (i, 0)),
            pl.BlockSpec((RB, 1), lambda i: (i, 0)),
            pl.BlockSpec((RB, 1), lambda i: (i, 0)),
            pl.BlockSpec((1, BT), lambda i: (0, 0)),
        ],
        out_specs=[
            pl.BlockSpec((BT, 1), lambda i: (0, 0)),
            pl.BlockSpec((BT, 2 * H), lambda i: (0, 0)),
        ],
        out_shape=[
            jax.ShapeDtypeStruct((BT, 1), _F32),
            jax.ShapeDtypeStruct((BT, 2 * H), _F32),
        ],
    )(feat, gidc, es, emax)


def _lstm_body(q_ref, run_ref, den_ref, hl_ref, cl_ref, wq_ref, wr_ref,
               wh_ref, bi_ref, bh_ref, ho_ref, co_ref):
    r = run_ref[...] / jnp.maximum(den_ref[...], 1e-30)
    g = (_dot(q_ref[...], wq_ref[...]) + _dot(r, wr_ref[...])
         + _dot(hl_ref[...], wh_ref[...]) + bi_ref[...] + bh_ref[...])
    gi = jax.nn.sigmoid(g[:, 0 * LAT:1 * LAT])
    gf = jax.nn.sigmoid(g[:, 1 * LAT:2 * LAT])
    gg = jnp.tanh(g[:, 2 * LAT:3 * LAT])
    go = jax.nn.sigmoid(g[:, 3 * LAT:4 * LAT])
    c2 = gf * cl_ref[...] + gi * gg
    ho_ref[...] = go * jnp.tanh(c2)
    co_ref[...] = c2


def _lstm(q, run, den, hl, cl, wq, wr, wh, bi, bh):
    return pl.pallas_call(
        _lstm_body,
        out_shape=[
            jax.ShapeDtypeStruct((BT, 2 * H), _F32),
            jax.ShapeDtypeStruct((BT, 2 * H), _F32),
        ],
    )(q, run, den, hl, cl, wq, wr, wh, bi, bh)


def _sp_body(q_ref, run_ref, den_ref, wq_ref, wr_ref, b_ref, a_ref, o_ref):
    r = run_ref[...] / jnp.maximum(den_ref[...], 1e-30)
    g = _dot(q_ref[...], wq_ref[...]) + _dot(r, wr_ref[...]) + b_ref[...]
    a = a_ref[0, 0]
    o_ref[...] = jnp.where(g >= 0, g, a * g)


def _sp(q, run, den, wq, wr, b, a):
    return pl.pallas_call(
        _sp_body,
        out_shape=jax.ShapeDtypeStruct((BT, RO), _F32),
    )(q, run, den, wq, wr, b, a)


def _lin_body(x_ref, w_ref, b_ref, a_ref, o_ref):
    y = _dot(x_ref[...], w_ref[...]) + b_ref[...]
    a = a_ref[0, 0]
    o_ref[...] = jnp.where(y >= 0, y, a * y)


def _linear_prelu(x, w, b, a):
    return pl.pallas_call(
        _lin_body,
        out_shape=jax.ShapeDtypeStruct((x.shape[0], w.shape[1]), _F32),
    )(x, w, b, a)


def _enc3_body(x_ref, w_ref, b_ref, eps_ref, mu_ref, lv_ref, lat_ref):
    y = _dot(x_ref[...], w_ref[...]) + b_ref[...]
    mu = jnp.clip(y[:, :LAT], -10.0, 10.0)
    lv = jnp.clip(y[:, LAT:], -10.0, 10.0)
    mu_ref[...] = mu
    lv_ref[...] = lv
    lat_ref[...] = mu + eps_ref[...] * jnp.exp(0.5 * lv)


def _enc3(x, w, b, eps):
    return pl.pallas_call(
        _enc3_body,
        out_shape=[
            jax.ShapeDtypeStruct((B, LAT), _F32),
            jax.ShapeDtypeStruct((B, LAT), _F32),
            jax.ShapeDtypeStruct((B, LAT), _F32),
        ],
    )(x, w, b, eps)


def _dec3_body(x_ref, w_ref, b_ref, o_ref):
    o_ref[...] = jnp.clip(_dot(x_ref[...], w_ref[...]) + b_ref[...], -10.0, 10.0)


def _dec3(x, w, b):
    return pl.pallas_call(
        _dec3_body,
        out_shape=jax.ShapeDtypeStruct((B, 128), _F32),
    )(x, w, b)


def kernel(r1_x, r1_e, r1_src, r1_dst, r1_gid, r2_x, r2_e, r2_src, r2_dst,
           r2_gid, pm_x, pm_e, pm_src, pm_dst, pm_gid, labels,
           pos_neg_sample, params):
    p = params
    f32 = _F32

    x = jnp.concatenate([r1_x, r2_x, pm_x], axis=0)
    ev = jnp.concatenate([r1_e, r2_e, pm_e], axis=0)
    src = jnp.concatenate(
        [r1_src, r2_src + N, pm_src + 2 * N]).astype(jnp.int32).reshape(NW, NCH, 128)
    dst = jnp.concatenate(
        [r1_dst, r2_dst + N, pm_dst + 2 * N]).astype(jnp.int32).reshape(16, NCS, 128)
    gidc = jnp.concatenate(
        [r1_gid, r2_gid + B, pm_gid + 2 * B]).astype(f32).reshape(NT, 1)

    w2 = p['edge_W'].reshape(DE * H, H)
    bmat = p['edge_b'].reshape(H, H)
    cb = p['conv_b'].reshape(1, H)
    gws = ([p['gru_Wih'][k * H:(k + 1) * H].T for k in range(3)]
           + [p['gru_Whh'][k * H:(k + 1) * H].T for k in range(3)])
    gbs = ([p['gru_bih'][k * H:(k + 1) * H].reshape(1, H) for k in range(3)]
           + [p['gru_bhh'][k * H:(k + 1) * H].reshape(1, H) for k in range(3)])

    x0 = _proj(x, p['proj_W'], p['proj_b'].reshape(1, H))
    h = x0
    for _ in range(3):
        hs = _sc_gather(h, src)
        m = _msg(hs, ev, w2, bmat)
        agg = _sc_scatter(m, dst)
        h = _gru(agg, h, cb, gws, gbs)

    feat = jnp.concatenate([x0, h], axis=1)

    lwq = p['lstm_Wih'][:, :2 * H].T
    lwr = p['lstm_Wih'][:, 2 * H:].T
    lwh = p['lstm_Whh'].T
    lbi = p['lstm_bih'].reshape(1, 8 * H)
    lbh = p['lstm_bhh'].reshape(1, 8 * H)
    q = jnp.zeros((BT, 2 * H), f32)
    run = jnp.zeros((BT, 2 * H), f32)
    den = jnp.ones((BT, 1), f32)
    hl = jnp.zeros((BT, 2 * H), f32)
    cl = jnp.zeros((BT, 2 * H), f32)
    for _ in range(3):
        hl, cl = _lstm(q, run, den, hl, cl, lwq, lwr, lwh, lbi, lbh)
        q = hl
        es, emax = _r1(feat, gidc, q)
        den, run = _r2(feat, gidc, es, emax)

    g = _sp(q, run, den, p['sp_W'][:2 * H], p['sp_W'][2 * H:],
            p['sp_b'].reshape(1, RO), p['sp_a'].reshape(1, 1))
    ge = g.reshape(NG, B, RO).transpose(1, 0, 2).reshape(B, NG * RO)

    pn = jnp.full((B, 1), pos_neg_sample, f32)
    ei = NG * RO + NC + 1
    eip = 3328
    xin = jnp.concatenate(
        [labels, ge, pn, jnp.zeros((B, eip - ei), f32)], axis=1)
    ew0 = jnp.concatenate(
        [p['enc_W0'], jnp.zeros((eip - ei, PH), f32)], axis=0)
    z = _linear_prelu(xin, ew0, p['enc_b0'].reshape(1, PH), p['enc_a0'].reshape(1, 1))
    z = _linear_prelu(z, p['enc_W1'], p['enc_b1'].reshape(1, PH), p['enc_a1'].reshape(1, 1))
    z = _linear_prelu(z, p['enc_W2'], p['enc_b2'].reshape(1, PH), p['enc_a2'].reshape(1, 1))
    eps = jax.random.normal(jax.random.key(42), (B, LAT), f32)
    mu, log_var, latent = _enc3(z, p['enc_W3'], p['enc_b3'].reshape(1, 2 * LAT), eps)

    di = NG * RO + LAT + 1
    dip = 3328
    yin = jnp.concatenate(
        [latent, ge, pn, jnp.zeros((B, dip - di), f32)], axis=1)
    dw0 = jnp.concatenate(
        [p['dec_W0'], jnp.zeros((dip - di, PH), f32)], axis=0)
    y = _linear_prelu(yin, dw0, p['dec_b0'].reshape(1, PH), p['dec_a0'].reshape(1, 1))
    y = _linear_prelu(y, p['dec_W1'], p['dec_b1'].reshape(1, PH), p['dec_a1'].reshape(1, 1))
    y = _linear_prelu(y, p['dec_W2'], p['dec_b2'].reshape(1, PH), p['dec_a2'].reshape(1, 1))
    dw3 = jnp.pad(p['dec_W3'], ((0, 0), (0, 128 - NC)))
    db3 = jnp.pad(p['dec_b3'], (0, 128 - NC)).reshape(1, 128)
    y = _dec3(y, dw3, db3)[:, :NC]
    return (y, mu, log_var)


# speedup vs baseline: 3.0000x; 1.1418x over previous
import functools

import jax
import jax.numpy as jnp
from jax import lax
from jax.experimental import pallas as pl
from jax.experimental.pallas import tpu as pltpu
from jax.experimental.pallas import tpu_sc as plsc

H = 64
DN = 128
DE = 16
RO = 1024
PH = 512
LAT = 128
NC = 100
B = 256
N = 8192
E = 16384
NG = 3
NT = NG * N
ET = NG * E
BT = NG * B

NW = 32
EPW = ET // NW
NCH = EPW // 128
RPS = NT // 16
HH = H // 4
EPS = ET // 16
NCS = EPS // 128

NB = 1024
EB = 512
RB = 512

_F32 = jnp.float32


def _dot(a, b):
    return jnp.dot(a, b, preferred_element_type=_F32)


def _sc_gather_body(h_hbm, idx_hbm, out_hbm, idx_v, rows_v, sem):
    wid = lax.axis_index("s") * 2 + lax.axis_index("c")
    pltpu.sync_copy(idx_hbm.at[wid], idx_v)
    cps = []
    for j in range(NCH):
        cps.append(
            pltpu.async_copy(h_hbm.at[idx_v.at[j]], rows_v.at[pl.ds(j * 128, 128)], sem)
        )
    for cp in cps:
        cp.wait()
    pltpu.sync_copy(rows_v, out_hbm.at[pl.ds(wid * EPW, EPW)])


def _sc_scatter_body(m_hbm, idx_hbm, out_hbm, idx_v, rows_v, zbuf, acc):
    c = lax.axis_index("c")
    s = lax.axis_index("s")

    def _zrow(i, carry):
        zbuf[i, pl.ds(0, 16)] = jnp.zeros((16,), _F32)
        return carry

    lax.fori_loop(0, 128, _zrow, 0)
    pltpu.sync_copy(idx_hbm.at[s], idx_v)
    for p in range(2):
        for j in range(RPS // 128):
            pltpu.sync_copy(zbuf, acc.at[pl.ds(s * RPS + j * 128, 128)])
        plsc.subcore_barrier()
        pltpu.sync_copy(
            m_hbm.at[pl.ds(s * EPS, EPS), pl.ds((2 * p + c) * HH, HH)], rows_v)
        for j in range(NCS):
            pltpu.sync_copy(rows_v.at[pl.ds(j * 128, 128)], acc.at[idx_v.at[j]],
                            add=True)
        plsc.subcore_barrier()
        pltpu.sync_copy(acc.at[pl.ds(s * RPS, RPS)],
                        out_hbm.at[pl.ds(s * RPS, RPS), pl.ds((2 * p + c) * HH, HH)])


@functools.lru_cache(maxsize=None)
def _sc_kernels():
    mesh = plsc.VectorSubcoreMesh(core_axis_name="c", subcore_axis_name="s")
    gather = pl.kernel(
        _sc_gather_body,
        out_type=jax.ShapeDtypeStruct((ET, H), _F32),
        mesh=mesh,
        compiler_params=pltpu.CompilerParams(use_tc_tiling_on_sc=False),
        scratch_types=[
            pltpu.VMEM((NCH, 128), jnp.int32),
            pltpu.VMEM((EPW, H), _F32),
            pltpu.SemaphoreType.DMA,
        ],
    )
    scatter = pl.kernel(
        _sc_scatter_body,
        out_type=jax.ShapeDtypeStruct((NT, H), _F32),
        mesh=mesh,
        compiler_params=pltpu.CompilerParams(use_tc_tiling_on_sc=False),
        scratch_types=[
            pltpu.VMEM((NCS, 128), jnp.int32),
            pltpu.VMEM((EPS, HH), _F32),
            pltpu.VMEM((128, HH), _F32),
            pltpu.VMEM_SHARED((NT, HH), _F32),
        ],
    )
    return gather, scatter


def _sc_gather(h, src):
    return _sc_kernels()[0](h, src)


def _sc_scatter(m, dst):
    return _sc_kernels()[1](m, dst)


def _proj_body(x_ref, w_ref, b_ref, o_ref):
    o_ref[...] = jnp.maximum(_dot(x_ref[...], w_ref[...]) + b_ref[...], 0.0)


def _proj(x, w, b):
    return pl.pallas_call(
        _proj_body,
        grid=(NT // NB,),
        in_specs=[
            pl.BlockSpec((NB, DN), lambda i: (i, 0)),
            pl.BlockSpec((DN, H), lambda i: (0, 0)),
            pl.BlockSpec((1, H), lambda i: (0, 0)),
        ],
        out_specs=pl.BlockSpec((NB, H), lambda i: (i, 0)),
        out_shape=jax.ShapeDtypeStruct((NT, H), _F32),
    )(x, w, b)


def _msg_body(hs_ref, evt_ref, w2_ref, bm_ref, o_ref):
    hst = jnp.transpose(hs_ref[...])
    evt = evt_ref[...]
    kt = (evt[:, None, :] * hst[None, :, :]).reshape(DE * H, EB)
    dn = (((0,), (0,)), ((), ()))
    o_ref[...] = (lax.dot_general(kt, w2_ref[...], dn, preferred_element_type=_F32)
                  + lax.dot_general(hst, bm_ref[...], dn, preferred_element_type=_F32))


def _msg(hs, evt, w2, bmat):
    return pl.pallas_call(
        _msg_body,
        grid=(ET // EB,),
        in_specs=[
            pl.BlockSpec((EB, H), lambda i: (i, 0)),
            pl.BlockSpec((DE, EB), lambda i: (0, i)),
            pl.BlockSpec((DE * H, H), lambda i: (0, 0)),
            pl.BlockSpec((H, H), lambda i: (0, 0)),
        ],
        out_specs=pl.BlockSpec((EB, H), lambda i: (i, 0)),
        out_shape=jax.ShapeDtypeStruct((ET, H), _F32),
    )(hs, evt, w2, bmat)


def _gru_body(ag_ref, h_ref, cb_ref, wir_ref, wiz_ref, win_ref,
              whr_ref, whz_ref, whn_ref, bir_ref, biz_ref, bin_ref,
              bhr_ref, bhz_ref, bhn_ref, o_ref):
    a = jnp.maximum(ag_ref[...] + cb_ref[...], 0.0)
    h = h_ref[...]
    r = jax.nn.sigmoid(_dot(a, wir_ref[...]) + bir_ref[...]
                       + _dot(h, whr_ref[...]) + bhr_ref[...])
    z = jax.nn.sigmoid(_dot(a, wiz_ref[...]) + biz_ref[...]
                       + _dot(h, whz_ref[...]) + bhz_ref[...])
    n = jnp.tanh(_dot(a, win_ref[...]) + bin_ref[...]
                 + r * (_dot(h, whn_ref[...]) + bhn_ref[...]))
    o_ref[...] = (1.0 - z) * n + z * h


def _gru(ag, h, cb, ws, bs):
    mat = pl.BlockSpec((H, H), lambda i: (0, 0))
    vec = pl.BlockSpec((1, H), lambda i: (0, 0))
    big = pl.BlockSpec((NB, H), lambda i: (i, 0))
    return pl.pallas_call(
        _gru_body,
        grid=(NT // NB,),
        in_specs=[big, big, vec] + [mat] * 6 + [vec] * 6,
        out_specs=big,
        out_shape=jax.ShapeDtypeStruct((NT, H), _F32),
    )(ag, h, cb, *ws, *bs)


def _ro_body(feat_ref, gid_ref, q_ref, mx_ref, den_ref, run_ref):
    i = pl.program_id(0)
    feat = feat_ref[...]
    gid = gid_ref[...]
    segs = lax.broadcasted_iota(jnp.int32, (RB, BT), 1).astype(_F32)
    mask = gid == segs
    qg = _dot(mask.astype(_F32), q_ref[...])
    es = jnp.sum(feat * qg, axis=1, keepdims=True)

    @pl.when(i == 0)
    def _():
        mx_ref[...] = jnp.full((1, BT), -1e30, _F32)
        den_ref[...] = jnp.zeros((BT, 1), _F32)
        run_ref[...] = jnp.zeros((BT, 2 * H), _F32)

    part = jnp.max(jnp.where(mask, es, -1e30), axis=0, keepdims=True)
    newmx = jnp.maximum(mx_ref[...], part)
    scale = jnp.transpose(jnp.exp(mx_ref[...] - newmx))
    mx_ref[...] = newmx
    emaxg = jnp.max(jnp.where(mask, newmx, -1e30), axis=1, keepdims=True)
    ex = jnp.exp(es - emaxg)
    exw = jnp.where(mask, ex, 0.0)
    dn = (((0,), (0,)), ((), ()))
    den_ref[...] = den_ref[...] * scale + lax.dot_general(
        exw, jnp.ones((RB, 1), _F32), dn, preferred_element_type=_F32)
    run_ref[...] = run_ref[...] * scale + lax.dot_general(
        exw, feat, dn, preferred_element_type=_F32)


def _ro(feat, gidc, q):
    return pl.pallas_call(
        _ro_body,
        grid=(NT // RB,),
        in_specs=[
            pl.BlockSpec((RB, 2 * H), lambda i: (i, 0)),
            pl.BlockSpec((RB, 1), lambda i: (i, 0)),
            pl.BlockSpec((BT, 2 * H), lambda i: (0, 0)),
        ],
        out_specs=[
            pl.BlockSpec((1, BT), lambda i: (0, 0)),
            pl.BlockSpec((BT, 1), lambda i: (0, 0)),
            pl.BlockSpec((BT, 2 * H), lambda i: (0, 0)),
        ],
        out_shape=[
            jax.ShapeDtypeStruct((1, BT), _F32),
            jax.ShapeDtypeStruct((BT, 1), _F32),
            jax.ShapeDtypeStruct((BT, 2 * H), _F32),
        ],
    )(feat, gidc, q)


def _lstm_body(q_ref, run_ref, den_ref, hl_ref, cl_ref, wq_ref, wr_ref,
               wh_ref, bi_ref, bh_ref, ho_ref, co_ref):
    r = run_ref[...] / jnp.maximum(den_ref[...], 1e-30)
    g = (_dot(q_ref[...], wq_ref[...]) + _dot(r, wr_ref[...])
         + _dot(hl_ref[...], wh_ref[...]) + bi_ref[...] + bh_ref[...])
    gi = jax.nn.sigmoid(g[:, 0 * LAT:1 * LAT])
    gf = jax.nn.sigmoid(g[:, 1 * LAT:2 * LAT])
    gg = jnp.tanh(g[:, 2 * LAT:3 * LAT])
    go = jax.nn.sigmoid(g[:, 3 * LAT:4 * LAT])
    c2 = gf * cl_ref[...] + gi * gg
    ho_ref[...] = go * jnp.tanh(c2)
    co_ref[...] = c2


def _lstm(q, run, den, hl, cl, wq, wr, wh, bi, bh):
    return pl.pallas_call(
        _lstm_body,
        out_shape=[
            jax.ShapeDtypeStruct((BT, 2 * H), _F32),
            jax.ShapeDtypeStruct((BT, 2 * H), _F32),
        ],
    )(q, run, den, hl, cl, wq, wr, wh, bi, bh)


def _sp_body(q_ref, run_ref, den_ref, wq_ref, wr_ref, b_ref, a_ref, o_ref):
    r = run_ref[...] / jnp.maximum(den_ref[...], 1e-30)
    g = _dot(q_ref[...], wq_ref[...]) + _dot(r, wr_ref[...]) + b_ref[...]
    a = a_ref[0, 0]
    o_ref[...] = jnp.where(g >= 0, g, a * g)


def _sp(q, run, den, wq, wr, b, a):
    return pl.pallas_call(
        _sp_body,
        out_shape=jax.ShapeDtypeStruct((BT, RO), _F32),
    )(q, run, den, wq, wr, b, a)


def _lin_body(x_ref, w_ref, b_ref, a_ref, o_ref):
    y = _dot(x_ref[...], w_ref[...]) + b_ref[...]
    a = a_ref[0, 0]
    o_ref[...] = jnp.where(y >= 0, y, a * y)


def _linear_prelu(x, w, b, a):
    return pl.pallas_call(
        _lin_body,
        out_shape=jax.ShapeDtypeStruct((x.shape[0], w.shape[1]), _F32),
    )(x, w, b, a)


def _enc3_body(x_ref, w_ref, b_ref, eps_ref, mu_ref, lv_ref, lat_ref):
    y = _dot(x_ref[...], w_ref[...]) + b_ref[...]
    mu = jnp.clip(y[:, :LAT], -10.0, 10.0)
    lv = jnp.clip(y[:, LAT:], -10.0, 10.0)
    mu_ref[...] = mu
    lv_ref[...] = lv
    lat_ref[...] = mu + eps_ref[...] * jnp.exp(0.5 * lv)


def _enc3(x, w, b, eps):
    return pl.pallas_call(
        _enc3_body,
        out_shape=[
            jax.ShapeDtypeStruct((B, LAT), _F32),
            jax.ShapeDtypeStruct((B, LAT), _F32),
            jax.ShapeDtypeStruct((B, LAT), _F32),
        ],
    )(x, w, b, eps)


def _dec3_body(x_ref, w_ref, b_ref, o_ref):
    o_ref[...] = jnp.clip(_dot(x_ref[...], w_ref[...]) + b_ref[...], -10.0, 10.0)


def _dec3(x, w, b):
    return pl.pallas_call(
        _dec3_body,
        out_shape=jax.ShapeDtypeStruct((B, 128), _F32),
    )(x, w, b)


def kernel(r1_x, r1_e, r1_src, r1_dst, r1_gid, r2_x, r2_e, r2_src, r2_dst,
           r2_gid, pm_x, pm_e, pm_src, pm_dst, pm_gid, labels,
           pos_neg_sample, params):
    p = params
    f32 = _F32

    x = jnp.concatenate([r1_x, r2_x, pm_x], axis=0)
    src = jnp.concatenate(
        [r1_src, r2_src + N, pm_src + 2 * N]).astype(jnp.int32).reshape(NW, NCH, 128)
    dst = jnp.concatenate(
        [r1_dst, r2_dst + N, pm_dst + 2 * N]).astype(jnp.int32).reshape(16, NCS, 128)
    gidc = jnp.concatenate(
        [r1_gid, r2_gid + B, pm_gid + 2 * B]).astype(f32).reshape(NT, 1)

    w2 = p['edge_W'].reshape(DE * H, H)
    evt = jnp.concatenate([r1_e, r2_e, pm_e], axis=0).T
    bmat = p['edge_b'].reshape(H, H)
    cb = p['conv_b'].reshape(1, H)
    gws = ([p['gru_Wih'][k * H:(k + 1) * H].T for k in range(3)]
           + [p['gru_Whh'][k * H:(k + 1) * H].T for k in range(3)])
    gbs = ([p['gru_bih'][k * H:(k + 1) * H].reshape(1, H) for k in range(3)]
           + [p['gru_bhh'][k * H:(k + 1) * H].reshape(1, H) for k in range(3)])

    x0 = _proj(x, p['proj_W'], p['proj_b'].reshape(1, H))
    h = x0
    for _ in range(3):
        hs = _sc_gather(h, src)
        m = _msg(hs, evt, w2, bmat)
        agg = _sc_scatter(m, dst)
        h = _gru(agg, h, cb, gws, gbs)

    feat = jnp.concatenate([x0, h], axis=1)

    lwq = p['lstm_Wih'][:, :2 * H].T
    lwr = p['lstm_Wih'][:, 2 * H:].T
    lwh = p['lstm_Whh'].T
    lbi = p['lstm_bih'].reshape(1, 8 * H)
    lbh = p['lstm_bhh'].reshape(1, 8 * H)
    q = jnp.zeros((BT, 2 * H), f32)
    run = jnp.zeros((BT, 2 * H), f32)
    den = jnp.ones((BT, 1), f32)
    hl = jnp.zeros((BT, 2 * H), f32)
    cl = jnp.zeros((BT, 2 * H), f32)
    for _ in range(3):
        hl, cl = _lstm(q, run, den, hl, cl, lwq, lwr, lwh, lbi, lbh)
        q = hl
        _, den, run = _ro(feat, gidc, q)

    g = _sp(q, run, den, p['sp_W'][:2 * H], p['sp_W'][2 * H:],
            p['sp_b'].reshape(1, RO), p['sp_a'].reshape(1, 1))
    ge = g.reshape(NG, B, RO).transpose(1, 0, 2).reshape(B, NG * RO)

    pn = jnp.full((B, 1), pos_neg_sample, f32)
    ei = NG * RO + NC + 1
    eip = 3328
    xin = jnp.concatenate(
        [labels, ge, pn, jnp.zeros((B, eip - ei), f32)], axis=1)
    ew0 = jnp.concatenate(
        [p['enc_W0'], jnp.zeros((eip - ei, PH), f32)], axis=0)
    z = _linear_prelu(xin, ew0, p['enc_b0'].reshape(1, PH), p['enc_a0'].reshape(1, 1))
    z = _linear_prelu(z, p['enc_W1'], p['enc_b1'].reshape(1, PH), p['enc_a1'].reshape(1, 1))
    z = _linear_prelu(z, p['enc_W2'], p['enc_b2'].reshape(1, PH), p['enc_a2'].reshape(1, 1))
    eps = jax.random.normal(jax.random.key(42), (B, LAT), f32)
    mu, log_var, latent = _enc3(z, p['enc_W3'], p['enc_b3'].reshape(1, 2 * LAT), eps)

    di = NG * RO + LAT + 1
    dip = 3328
    yin = jnp.concatenate(
        [latent, ge, pn, jnp.zeros((B, dip - di), f32)], axis=1)
    dw0 = jnp.concatenate(
        [p['dec_W0'], jnp.zeros((dip - di, PH), f32)], axis=0)
    y = _linear_prelu(yin, dw0, p['dec_b0'].reshape(1, PH), p['dec_a0'].reshape(1, 1))
    y = _linear_prelu(y, p['dec_W1'], p['dec_b1'].reshape(1, PH), p['dec_a1'].reshape(1, 1))
    y = _linear_prelu(y, p['dec_W2'], p['dec_b2'].reshape(1, PH), p['dec_a2'].reshape(1, 1))
    dw3 = jnp.pad(p['dec_W3'], ((0, 0), (0, 128 - NC)))
    db3 = jnp.pad(p['dec_b3'], (0, 128 - NC)).reshape(1, 128)
    y = _dec3(y, dw3, db3)[:, :NC]
    return (y, mu, log_var)


# speedup vs baseline: 4.2823x; 1.4274x over previous
import functools

import jax
import jax.numpy as jnp
from jax import lax
from jax.experimental import pallas as pl
from jax.experimental.pallas import tpu as pltpu
from jax.experimental.pallas import tpu_sc as plsc

H = 64
DN = 128
DE = 16
RO = 1024
PH = 512
LAT = 128
NC = 100
B = 256
N = 8192
E = 16384
NG = 3
NT = NG * N
ET = NG * E
BT = NG * B

NW = 32
EPW = ET // NW
NCH = EPW // 128
RPS = NT // 16
HH = H // 4
EPS = ET // 16
NCS = EPS // 128

NB = 2048
EB = 1024
RB = 1024

_F32 = jnp.float32


def _dot(a, b):
    return jnp.dot(a, b, preferred_element_type=_F32)


def _sc_gather_body(h_hbm, idx_hbm, out_hbm, idx_v, rows_v, sem):
    wid = lax.axis_index("s") * 2 + lax.axis_index("c")
    pltpu.sync_copy(idx_hbm.at[wid], idx_v)
    cps = []
    for j in range(NCH):
        cps.append(
            pltpu.async_copy(h_hbm.at[idx_v.at[j]], rows_v.at[pl.ds(j * 128, 128)], sem)
        )
    for cp in cps:
        cp.wait()
    pltpu.sync_copy(rows_v, out_hbm.at[pl.ds(wid * EPW, EPW)])


def _sc_scatter_body(m_hbm, idx_hbm, out_hbm, idx_v, rows_v, zbuf, acc):
    c = lax.axis_index("c")
    s = lax.axis_index("s")

    def _zrow(i, carry):
        zbuf[i, pl.ds(0, 16)] = jnp.zeros((16,), _F32)
        return carry

    lax.fori_loop(0, 128, _zrow, 0)
    pltpu.sync_copy(idx_hbm.at[s], idx_v)
    for p in range(2):
        for j in range(RPS // 128):
            pltpu.sync_copy(zbuf, acc.at[pl.ds(s * RPS + j * 128, 128)])
        plsc.subcore_barrier()
        pltpu.sync_copy(
            m_hbm.at[pl.ds(s * EPS, EPS), pl.ds((2 * p + c) * HH, HH)], rows_v)
        for j in range(NCS):
            pltpu.sync_copy(rows_v.at[pl.ds(j * 128, 128)], acc.at[idx_v.at[j]],
                            add=True)
        plsc.subcore_barrier()
        pltpu.sync_copy(acc.at[pl.ds(s * RPS, RPS)],
                        out_hbm.at[pl.ds(s * RPS, RPS), pl.ds((2 * p + c) * HH, HH)])


@functools.lru_cache(maxsize=None)
def _sc_kernels():
    mesh = plsc.VectorSubcoreMesh(core_axis_name="c", subcore_axis_name="s")
    gather = pl.kernel(
        _sc_gather_body,
        out_type=jax.ShapeDtypeStruct((ET, H), _F32),
        mesh=mesh,
        compiler_params=pltpu.CompilerParams(use_tc_tiling_on_sc=False),
        scratch_types=[
            pltpu.VMEM((NCH, 128), jnp.int32),
            pltpu.VMEM((EPW, H), _F32),
            pltpu.SemaphoreType.DMA,
        ],
    )
    scatter = pl.kernel(
        _sc_scatter_body,
        out_type=jax.ShapeDtypeStruct((NT, H), _F32),
        mesh=mesh,
        compiler_params=pltpu.CompilerParams(use_tc_tiling_on_sc=False),
        scratch_types=[
            pltpu.VMEM((NCS, 128), jnp.int32),
            pltpu.VMEM((EPS, HH), _F32),
            pltpu.VMEM((128, HH), _F32),
            pltpu.VMEM_SHARED((NT, HH), _F32),
        ],
    )
    return gather, scatter


def _sc_gather(h, src):
    return _sc_kernels()[0](h, src)


def _sc_scatter(m, dst):
    return _sc_kernels()[1](m, dst)


def _proj_body(x_ref, w_ref, b_ref, o_ref):
    o_ref[...] = jnp.maximum(_dot(x_ref[...], w_ref[...]) + b_ref[...], 0.0)


def _proj(x, w, b):
    return pl.pallas_call(
        _proj_body,
        grid=(NT // NB,),
        in_specs=[
            pl.BlockSpec((NB, DN), lambda i: (i, 0)),
            pl.BlockSpec((DN, H), lambda i: (0, 0)),
            pl.BlockSpec((1, H), lambda i: (0, 0)),
        ],
        out_specs=pl.BlockSpec((NB, H), lambda i: (i, 0)),
        out_shape=jax.ShapeDtypeStruct((NT, H), _F32),
    )(x, w, b)


def _msg_body(hs_ref, evt_ref, w2_ref, bm_ref, o_ref):
    hst = jnp.transpose(hs_ref[...])
    evt = evt_ref[...]
    kt = (evt[:, None, :] * hst[None, :, :]).reshape(DE * H, EB)
    dn = (((0,), (0,)), ((), ()))
    o_ref[...] = (lax.dot_general(kt, w2_ref[...], dn, preferred_element_type=_F32)
                  + lax.dot_general(hst, bm_ref[...], dn, preferred_element_type=_F32))


def _msg(hs, evt, w2, bmat):
    return pl.pallas_call(
        _msg_body,
        grid=(ET // EB,),
        in_specs=[
            pl.BlockSpec((EB, H), lambda i: (i, 0)),
            pl.BlockSpec((DE, EB), lambda i: (0, i)),
            pl.BlockSpec((DE * H, H), lambda i: (0, 0)),
            pl.BlockSpec((H, H), lambda i: (0, 0)),
        ],
        out_specs=pl.BlockSpec((EB, H), lambda i: (i, 0)),
        out_shape=jax.ShapeDtypeStruct((ET, H), _F32),
    )(hs, evt, w2, bmat)


def _gru_body(ag_ref, h_ref, cb_ref, wir_ref, wiz_ref, win_ref,
              whr_ref, whz_ref, whn_ref, bir_ref, biz_ref, bin_ref,
              bhr_ref, bhz_ref, bhn_ref, o_ref):
    a = jnp.maximum(ag_ref[...] + cb_ref[...], 0.0)
    h = h_ref[...]
    r = jax.nn.sigmoid(_dot(a, wir_ref[...]) + bir_ref[...]
                       + _dot(h, whr_ref[...]) + bhr_ref[...])
    z = jax.nn.sigmoid(_dot(a, wiz_ref[...]) + biz_ref[...]
                       + _dot(h, whz_ref[...]) + bhz_ref[...])
    n = jnp.tanh(_dot(a, win_ref[...]) + bin_ref[...]
                 + r * (_dot(h, whn_ref[...]) + bhn_ref[...]))
    o_ref[...] = (1.0 - z) * n + z * h


def _gru(ag, h, cb, ws, bs):
    mat = pl.BlockSpec((H, H), lambda i: (0, 0))
    vec = pl.BlockSpec((1, H), lambda i: (0, 0))
    big = pl.BlockSpec((NB, H), lambda i: (i, 0))
    return pl.pallas_call(
        _gru_body,
        grid=(NT // NB,),
        in_specs=[big, big, vec] + [mat] * 6 + [vec] * 6,
        out_specs=big,
        out_shape=jax.ShapeDtypeStruct((NT, H), _F32),
    )(ag, h, cb, *ws, *bs)


def _ro_body(feat_ref, gid_ref, q_ref, mx_ref, den_ref, run_ref):
    i = pl.program_id(0)
    feat = feat_ref[...]
    gid = gid_ref[...]
    segs = lax.broadcasted_iota(jnp.int32, (RB, BT), 1).astype(_F32)
    mask = gid == segs
    qg = _dot(mask.astype(_F32), q_ref[...])
    es = jnp.sum(feat * qg, axis=1, keepdims=True)

    @pl.when(i == 0)
    def _():
        mx_ref[...] = jnp.full((1, BT), -1e30, _F32)
        den_ref[...] = jnp.zeros((BT, 1), _F32)
        run_ref[...] = jnp.zeros((BT, 2 * H), _F32)

    part = jnp.max(jnp.where(mask, es, -1e30), axis=0, keepdims=True)
    newmx = jnp.maximum(mx_ref[...], part)
    scale = jnp.transpose(jnp.exp(mx_ref[...] - newmx))
    mx_ref[...] = newmx
    emaxg = jnp.max(jnp.where(mask, newmx, -1e30), axis=1, keepdims=True)
    ex = jnp.exp(es - emaxg)
    exw = jnp.where(mask, ex, 0.0)
    dn = (((0,), (0,)), ((), ()))
    den_ref[...] = den_ref[...] * scale + lax.dot_general(
        exw, jnp.ones((RB, 1), _F32), dn, preferred_element_type=_F32)
    run_ref[...] = run_ref[...] * scale + lax.dot_general(
        exw, feat, dn, preferred_element_type=_F32)


def _ro(feat, gidc, q):
    return pl.pallas_call(
        _ro_body,
        grid=(NT // RB,),
        in_specs=[
            pl.BlockSpec((RB, 2 * H), lambda i: (i, 0)),
            pl.BlockSpec((RB, 1), lambda i: (i, 0)),
            pl.BlockSpec((BT, 2 * H), lambda i: (0, 0)),
        ],
        out_specs=[
            pl.BlockSpec((1, BT), lambda i: (0, 0)),
            pl.BlockSpec((BT, 1), lambda i: (0, 0)),
            pl.BlockSpec((BT, 2 * H), lambda i: (0, 0)),
        ],
        out_shape=[
            jax.ShapeDtypeStruct((1, BT), _F32),
            jax.ShapeDtypeStruct((BT, 1), _F32),
            jax.ShapeDtypeStruct((BT, 2 * H), _F32),
        ],
    )(feat, gidc, q)


def _lstm_body(q_ref, run_ref, den_ref, hl_ref, cl_ref, wq_ref, wr_ref,
               wh_ref, bi_ref, bh_ref, ho_ref, co_ref):
    r = run_ref[...] / jnp.maximum(den_ref[...], 1e-30)
    g = (_dot(q_ref[...], wq_ref[...]) + _dot(r, wr_ref[...])
         + _dot(hl_ref[...], wh_ref[...]) + bi_ref[...] + bh_ref[...])
    gi = jax.nn.sigmoid(g[:, 0 * LAT:1 * LAT])
    gf = jax.nn.sigmoid(g[:, 1 * LAT:2 * LAT])
    gg = jnp.tanh(g[:, 2 * LAT:3 * LAT])
    go = jax.nn.sigmoid(g[:, 3 * LAT:4 * LAT])
    c2 = gf * cl_ref[...] + gi * gg
    ho_ref[...] = go * jnp.tanh(c2)
    co_ref[...] = c2


def _lstm(q, run, den, hl, cl, wq, wr, wh, bi, bh):
    return pl.pallas_call(
        _lstm_body,
        out_shape=[
            jax.ShapeDtypeStruct((BT, 2 * H), _F32),
            jax.ShapeDtypeStruct((BT, 2 * H), _F32),
        ],
    )(q, run, den, hl, cl, wq, wr, wh, bi, bh)


def _sp_body(q_ref, run_ref, den_ref, wq_ref, wr_ref, b_ref, a_ref, o_ref):
    r = run_ref[...] / jnp.maximum(den_ref[...], 1e-30)
    g = _dot(q_ref[...], wq_ref[...]) + _dot(r, wr_ref[...]) + b_ref[...]
    a = a_ref[0, 0]
    o_ref[...] = jnp.where(g >= 0, g, a * g)


def _sp(q, run, den, wq, wr, b, a):
    return pl.pallas_call(
        _sp_body,
        out_shape=jax.ShapeDtypeStruct((BT, RO), _F32),
    )(q, run, den, wq, wr, b, a)


def _lin_body(x_ref, w_ref, b_ref, a_ref, o_ref):
    y = _dot(x_ref[...], w_ref[...]) + b_ref[...]
    a = a_ref[0, 0]
    o_ref[...] = jnp.where(y >= 0, y, a * y)


def _linear_prelu(x, w, b, a):
    return pl.pallas_call(
        _lin_body,
        out_shape=jax.ShapeDtypeStruct((x.shape[0], w.shape[1]), _F32),
    )(x, w, b, a)


def _enc3_body(x_ref, w_ref, b_ref, eps_ref, mu_ref, lv_ref, lat_ref):
    y = _dot(x_ref[...], w_ref[...]) + b_ref[...]
    mu = jnp.clip(y[:, :LAT], -10.0, 10.0)
    lv = jnp.clip(y[:, LAT:], -10.0, 10.0)
    mu_ref[...] = mu
    lv_ref[...] = lv
    lat_ref[...] = mu + eps_ref[...] * jnp.exp(0.5 * lv)


def _enc3(x, w, b, eps):
    return pl.pallas_call(
        _enc3_body,
        out_shape=[
            jax.ShapeDtypeStruct((B, LAT), _F32),
            jax.ShapeDtypeStruct((B, LAT), _F32),
            jax.ShapeDtypeStruct((B, LAT), _F32),
        ],
    )(x, w, b, eps)


def _dec3_body(x_ref, w_ref, b_ref, o_ref):
    o_ref[...] = jnp.clip(_dot(x_ref[...], w_ref[...]) + b_ref[...], -10.0, 10.0)


def _dec3(x, w, b):
    return pl.pallas_call(
        _dec3_body,
        out_shape=jax.ShapeDtypeStruct((B, 128), _F32),
    )(x, w, b)


def kernel(r1_x, r1_e, r1_src, r1_dst, r1_gid, r2_x, r2_e, r2_src, r2_dst,
           r2_gid, pm_x, pm_e, pm_src, pm_dst, pm_gid, labels,
           pos_neg_sample, params):
    p = params
    f32 = _F32

    x = jnp.concatenate([r1_x, r2_x, pm_x], axis=0)
    src = jnp.concatenate(
        [r1_src, r2_src + N, pm_src + 2 * N]).astype(jnp.int32).reshape(NW, NCH, 128)
    dst = jnp.concatenate(
        [r1_dst, r2_dst + N, pm_dst + 2 * N]).astype(jnp.int32).reshape(16, NCS, 128)
    gidc = jnp.concatenate(
        [r1_gid, r2_gid + B, pm_gid + 2 * B]).astype(f32).reshape(NT, 1)

    w2 = p['edge_W'].reshape(DE * H, H)
    evt = jnp.concatenate([r1_e, r2_e, pm_e], axis=0).T
    bmat = p['edge_b'].reshape(H, H)
    cb = p['conv_b'].reshape(1, H)
    gws = ([p['gru_Wih'][k * H:(k + 1) * H].T for k in range(3)]
           + [p['gru_Whh'][k * H:(k + 1) * H].T for k in range(3)])
    gbs = ([p['gru_bih'][k * H:(k + 1) * H].reshape(1, H) for k in range(3)]
           + [p['gru_bhh'][k * H:(k + 1) * H].reshape(1, H) for k in range(3)])

    x0 = _proj(x, p['proj_W'], p['proj_b'].reshape(1, H))
    h = x0
    for _ in range(3):
        hs = _sc_gather(h, src)
        m = _msg(hs, evt, w2, bmat)
        agg = _sc_scatter(m, dst)
        h = _gru(agg, h, cb, gws, gbs)

    feat = jnp.concatenate([x0, h], axis=1)

    lwq = p['lstm_Wih'][:, :2 * H].T
    lwr = p['lstm_Wih'][:, 2 * H:].T
    lwh = p['lstm_Whh'].T
    lbi = p['lstm_bih'].reshape(1, 8 * H)
    lbh = p['lstm_bhh'].reshape(1, 8 * H)
    q = jnp.zeros((BT, 2 * H), f32)
    run = jnp.zeros((BT, 2 * H), f32)
    den = jnp.ones((BT, 1), f32)
    hl = jnp.zeros((BT, 2 * H), f32)
    cl = jnp.zeros((BT, 2 * H), f32)
    for _ in range(3):
        hl, cl = _lstm(q, run, den, hl, cl, lwq, lwr, lwh, lbi, lbh)
        q = hl
        _, den, run = _ro(feat, gidc, q)

    g = _sp(q, run, den, p['sp_W'][:2 * H], p['sp_W'][2 * H:],
            p['sp_b'].reshape(1, RO), p['sp_a'].reshape(1, 1))
    ge = g.reshape(NG, B, RO).transpose(1, 0, 2).reshape(B, NG * RO)

    pn = jnp.full((B, 1), pos_neg_sample, f32)
    ei = NG * RO + NC + 1
    eip = 3328
    xin = jnp.concatenate(
        [labels, ge, pn, jnp.zeros((B, eip - ei), f32)], axis=1)
    ew0 = jnp.concatenate(
        [p['enc_W0'], jnp.zeros((eip - ei, PH), f32)], axis=0)
    z = _linear_prelu(xin, ew0, p['enc_b0'].reshape(1, PH), p['enc_a0'].reshape(1, 1))
    z = _linear_prelu(z, p['enc_W1'], p['enc_b1'].reshape(1, PH), p['enc_a1'].reshape(1, 1))
    z = _linear_prelu(z, p['enc_W2'], p['enc_b2'].reshape(1, PH), p['enc_a2'].reshape(1, 1))
    eps = jax.random.normal(jax.random.key(42), (B, LAT), f32)
    mu, log_var, latent = _enc3(z, p['enc_W3'], p['enc_b3'].reshape(1, 2 * LAT), eps)

    di = NG * RO + LAT + 1
    dip = 3328
    yin = jnp.concatenate(
        [latent, ge, pn, jnp.zeros((B, dip - di), f32)], axis=1)
    dw0 = jnp.concatenate(
        [p['dec_W0'], jnp.zeros((dip - di, PH), f32)], axis=0)
    y = _linear_prelu(yin, dw0, p['dec_b0'].reshape(1, PH), p['dec_a0'].reshape(1, 1))
    y = _linear_prelu(y, p['dec_W1'], p['dec_b1'].reshape(1, PH), p['dec_a1'].reshape(1, 1))
    y = _linear_prelu(y, p['dec_W2'], p['dec_b2'].reshape(1, PH), p['dec_a2'].reshape(1, 1))
    dw3 = jnp.pad(p['dec_W3'], ((0, 0), (0, 128 - NC)))
    db3 = jnp.pad(p['dec_b3'], (0, 128 - NC)).reshape(1, 128)
    y = _dec3(y, dw3, db3)[:, :NC]
    return (y, mu, log_var)


# speedup vs baseline: 4.6429x; 1.0842x over previous
import functools

import jax
import jax.numpy as jnp
from jax import lax
from jax.experimental import pallas as pl
from jax.experimental.pallas import tpu as pltpu
from jax.experimental.pallas import tpu_sc as plsc

H = 64
DN = 128
DE = 16
RO = 1024
PH = 512
LAT = 128
NC = 100
B = 256
N = 8192
E = 16384
NG = 3
NT = NG * N
ET = NG * E
BT = NG * B

NW = 32
EPW = ET // NW
NCH = EPW // 128
RPS = NT // 16
HH = H // 4
EPS = ET // 16
NCS = EPS // 128

NB = 2048
EB = 1024
RB = 1024

_F32 = jnp.float32


def _dot(a, b):
    return jnp.dot(a, b, preferred_element_type=_F32)


def _sc_gather_body(f_hbm, idx_hbm, out_hbm, idx_v, rows_v, sem):
    wid = lax.axis_index("s") * 2 + lax.axis_index("c")
    pltpu.sync_copy(idx_hbm.at[wid], idx_v)
    for r in range(2):
        cps = []
        for j in range(NCH // 2):
            cps.append(
                pltpu.async_copy(f_hbm.at[idx_v.at[r * (NCH // 2) + j]],
                                 rows_v.at[pl.ds(j * 128, 128)], sem)
            )
        for cp in cps:
            cp.wait()
        pltpu.sync_copy(
            rows_v, out_hbm.at[pl.ds(wid * EPW + r * (EPW // 2), EPW // 2)])


def _sc_scatter_body(m_hbm, idx_hbm, out_hbm, idx_v, rows_v, zbuf, acc):
    c = lax.axis_index("c")
    s = lax.axis_index("s")

    def _zrow(i, carry):
        zbuf[i, pl.ds(0, 16)] = jnp.zeros((16,), _F32)
        return carry

    lax.fori_loop(0, 128, _zrow, 0)
    pltpu.sync_copy(idx_hbm.at[s], idx_v)
    for p in range(2):
        for j in range(RPS // 128):
            pltpu.sync_copy(zbuf, acc.at[pl.ds(s * RPS + j * 128, 128)])
        plsc.subcore_barrier()
        pltpu.sync_copy(
            m_hbm.at[pl.ds(s * EPS, EPS), pl.ds((2 * p + c) * HH, HH)], rows_v)
        for j in range(NCS):
            pltpu.sync_copy(rows_v.at[pl.ds(j * 128, 128)], acc.at[idx_v.at[j]],
                            add=True)
        plsc.subcore_barrier()
        pltpu.sync_copy(acc.at[pl.ds(s * RPS, RPS)],
                        out_hbm.at[pl.ds(s * RPS, RPS), pl.ds((2 * p + c) * HH, HH)])


@functools.lru_cache(maxsize=None)
def _sc_kernels():
    mesh = plsc.VectorSubcoreMesh(core_axis_name="c", subcore_axis_name="s")
    gather = pl.kernel(
        _sc_gather_body,
        out_type=jax.ShapeDtypeStruct((ET, 2 * H), _F32),
        mesh=mesh,
        scratch_types=[
            pltpu.VMEM((NCH, 128), jnp.int32),
            pltpu.VMEM((EPW // 2, 2 * H), _F32),
            pltpu.SemaphoreType.DMA,
        ],
    )
    scatter = pl.kernel(
        _sc_scatter_body,
        out_type=jax.ShapeDtypeStruct((NT, H), _F32),
        mesh=mesh,
        compiler_params=pltpu.CompilerParams(use_tc_tiling_on_sc=False),
        scratch_types=[
            pltpu.VMEM((NCS, 128), jnp.int32),
            pltpu.VMEM((EPS, HH), _F32),
            pltpu.VMEM((128, HH), _F32),
            pltpu.VMEM_SHARED((NT, HH), _F32),
        ],
    )
    return gather, scatter


def _sc_gather(h, src):
    return _sc_kernels()[0](h, src)


def _sc_scatter(m, dst):
    return _sc_kernels()[1](m, dst)


def _proj_body(x_ref, w_ref, b_ref, o_ref):
    y = jnp.maximum(_dot(x_ref[...], w_ref[...]) + b_ref[...], 0.0)
    o_ref[...] = jnp.concatenate([y, y], axis=1)


def _proj(x, w, b):
    return pl.pallas_call(
        _proj_body,
        grid=(NT // NB,),
        in_specs=[
            pl.BlockSpec((NB, DN), lambda i: (i, 0)),
            pl.BlockSpec((DN, H), lambda i: (0, 0)),
            pl.BlockSpec((1, H), lambda i: (0, 0)),
        ],
        out_specs=pl.BlockSpec((NB, 2 * H), lambda i: (i, 0)),
        out_shape=jax.ShapeDtypeStruct((NT, 2 * H), _F32),
    )(x, w, b)


def _msg_body(hs_ref, evt_ref, w2_ref, bm_ref, o_ref):
    hst = jnp.transpose(hs_ref[...])[H:, :]
    evt = evt_ref[...]
    kt = (evt[:, None, :] * hst[None, :, :]).reshape(DE * H, EB)
    dn = (((0,), (0,)), ((), ()))
    o_ref[...] = (lax.dot_general(kt, w2_ref[...], dn, preferred_element_type=_F32)
                  + lax.dot_general(hst, bm_ref[...], dn, preferred_element_type=_F32))


def _msg(hs, evt, w2, bmat):
    return pl.pallas_call(
        _msg_body,
        grid=(ET // EB,),
        in_specs=[
            pl.BlockSpec((EB, 2 * H), lambda i: (i, 0)),
            pl.BlockSpec((DE, EB), lambda i: (0, i)),
            pl.BlockSpec((DE * H, H), lambda i: (0, 0)),
            pl.BlockSpec((H, H), lambda i: (0, 0)),
        ],
        out_specs=pl.BlockSpec((EB, H), lambda i: (i, 0)),
        out_shape=jax.ShapeDtypeStruct((ET, H), _F32),
    )(hs, evt, w2, bmat)


def _gru_body(ag_ref, f_ref, cb_ref, wir_ref, wiz_ref, win_ref,
              whr_ref, whz_ref, whn_ref, bir_ref, biz_ref, bin_ref,
              bhr_ref, bhz_ref, bhn_ref, o_ref):
    a = jnp.maximum(ag_ref[...] + cb_ref[...], 0.0)
    f = f_ref[...]
    h = f[:, H:]
    r = jax.nn.sigmoid(_dot(a, wir_ref[...]) + bir_ref[...]
                       + _dot(h, whr_ref[...]) + bhr_ref[...])
    z = jax.nn.sigmoid(_dot(a, wiz_ref[...]) + biz_ref[...]
                       + _dot(h, whz_ref[...]) + bhz_ref[...])
    n = jnp.tanh(_dot(a, win_ref[...]) + bin_ref[...]
                 + r * (_dot(h, whn_ref[...]) + bhn_ref[...]))
    o_ref[...] = jnp.concatenate([f[:, :H], (1.0 - z) * n + z * h], axis=1)


def _gru(ag, f, cb, ws, bs):
    mat = pl.BlockSpec((H, H), lambda i: (0, 0))
    vec = pl.BlockSpec((1, H), lambda i: (0, 0))
    big = pl.BlockSpec((NB, H), lambda i: (i, 0))
    wide = pl.BlockSpec((NB, 2 * H), lambda i: (i, 0))
    return pl.pallas_call(
        _gru_body,
        grid=(NT // NB,),
        in_specs=[big, wide, vec] + [mat] * 6 + [vec] * 6,
        out_specs=wide,
        out_shape=jax.ShapeDtypeStruct((NT, 2 * H), _F32),
    )(ag, f, cb, *ws, *bs)


def _ro_body(feat_ref, gid_ref, q_ref, mx_ref, den_ref, run_ref):
    i = pl.program_id(0)
    feat = feat_ref[...]
    gid = gid_ref[...]
    segs = lax.broadcasted_iota(jnp.int32, (RB, BT), 1).astype(_F32)
    mask = gid == segs
    qg = _dot(mask.astype(_F32), q_ref[...])
    es = jnp.sum(feat * qg, axis=1, keepdims=True)

    @pl.when(i == 0)
    def _():
        mx_ref[...] = jnp.full((1, BT), -1e30, _F32)
        den_ref[...] = jnp.zeros((BT, 1), _F32)
        run_ref[...] = jnp.zeros((BT, 2 * H), _F32)

    part = jnp.max(jnp.where(mask, es, -1e30), axis=0, keepdims=True)
    newmx = jnp.maximum(mx_ref[...], part)
    scale = jnp.transpose(jnp.exp(mx_ref[...] - newmx))
    mx_ref[...] = newmx
    emaxg = jnp.max(jnp.where(mask, newmx, -1e30), axis=1, keepdims=True)
    ex = jnp.exp(es - emaxg)
    exw = jnp.where(mask, ex, 0.0)
    dn = (((0,), (0,)), ((), ()))
    den_ref[...] = den_ref[...] * scale + lax.dot_general(
        exw, jnp.ones((RB, 1), _F32), dn, preferred_element_type=_F32)
    run_ref[...] = run_ref[...] * scale + lax.dot_general(
        exw, feat, dn, preferred_element_type=_F32)


def _ro(feat, gidc, q):
    return pl.pallas_call(
        _ro_body,
        grid=(NT // RB,),
        in_specs=[
            pl.BlockSpec((RB, 2 * H), lambda i: (i, 0)),
            pl.BlockSpec((RB, 1), lambda i: (i, 0)),
            pl.BlockSpec((BT, 2 * H), lambda i: (0, 0)),
        ],
        out_specs=[
            pl.BlockSpec((1, BT), lambda i: (0, 0)),
            pl.BlockSpec((BT, 1), lambda i: (0, 0)),
            pl.BlockSpec((BT, 2 * H), lambda i: (0, 0)),
        ],
        out_shape=[
            jax.ShapeDtypeStruct((1, BT), _F32),
            jax.ShapeDtypeStruct((BT, 1), _F32),
            jax.ShapeDtypeStruct((BT, 2 * H), _F32),
        ],
    )(feat, gidc, q)


def _lstm_body(q_ref, run_ref, den_ref, hl_ref, cl_ref, wq_ref, wr_ref,
               wh_ref, bi_ref, bh_ref, ho_ref, co_ref):
    r = run_ref[...] / jnp.maximum(den_ref[...], 1e-30)
    g = (_dot(q_ref[...], wq_ref[...]) + _dot(r, wr_ref[...])
         + _dot(hl_ref[...], wh_ref[...]) + bi_ref[...] + bh_ref[...])
    gi = jax.nn.sigmoid(g[:, 0 * LAT:1 * LAT])
    gf = jax.nn.sigmoid(g[:, 1 * LAT:2 * LAT])
    gg = jnp.tanh(g[:, 2 * LAT:3 * LAT])
    go = jax.nn.sigmoid(g[:, 3 * LAT:4 * LAT])
    c2 = gf * cl_ref[...] + gi * gg
    ho_ref[...] = go * jnp.tanh(c2)
    co_ref[...] = c2


def _lstm(q, run, den, hl, cl, wq, wr, wh, bi, bh):
    return pl.pallas_call(
        _lstm_body,
        out_shape=[
            jax.ShapeDtypeStruct((BT, 2 * H), _F32),
            jax.ShapeDtypeStruct((BT, 2 * H), _F32),
        ],
    )(q, run, den, hl, cl, wq, wr, wh, bi, bh)


def _sp_body(q_ref, run_ref, den_ref, wq_ref, wr_ref, b_ref, a_ref, o_ref):
    r = run_ref[...] / jnp.maximum(den_ref[...], 1e-30)
    g = _dot(q_ref[...], wq_ref[...]) + _dot(r, wr_ref[...]) + b_ref[...]
    a = a_ref[0, 0]
    o_ref[...] = jnp.where(g >= 0, g, a * g)


def _sp(q, run, den, wq, wr, b, a):
    return pl.pallas_call(
        _sp_body,
        out_shape=jax.ShapeDtypeStruct((BT, RO), _F32),
    )(q, run, den, wq, wr, b, a)


def _lin_body(x_ref, w_ref, b_ref, a_ref, o_ref):
    y = _dot(x_ref[...], w_ref[...]) + b_ref[...]
    a = a_ref[0, 0]
    o_ref[...] = jnp.where(y >= 0, y, a * y)


def _linear_prelu(x, w, b, a):
    return pl.pallas_call(
        _lin_body,
        out_shape=jax.ShapeDtypeStruct((x.shape[0], w.shape[1]), _F32),
    )(x, w, b, a)


def _enc3_body(x_ref, w_ref, b_ref, eps_ref, mu_ref, lv_ref, lat_ref):
    y = _dot(x_ref[...], w_ref[...]) + b_ref[...]
    mu = jnp.clip(y[:, :LAT], -10.0, 10.0)
    lv = jnp.clip(y[:, LAT:], -10.0, 10.0)
    mu_ref[...] = mu
    lv_ref[...] = lv
    lat_ref[...] = mu + eps_ref[...] * jnp.exp(0.5 * lv)


def _enc3(x, w, b, eps):
    return pl.pallas_call(
        _enc3_body,
        out_shape=[
            jax.ShapeDtypeStruct((B, LAT), _F32),
            jax.ShapeDtypeStruct((B, LAT), _F32),
            jax.ShapeDtypeStruct((B, LAT), _F32),
        ],
    )(x, w, b, eps)


def _dec3_body(x_ref, w_ref, b_ref, o_ref):
    o_ref[...] = jnp.clip(_dot(x_ref[...], w_ref[...]) + b_ref[...], -10.0, 10.0)


def _dec3(x, w, b):
    return pl.pallas_call(
        _dec3_body,
        out_shape=jax.ShapeDtypeStruct((B, 128), _F32),
    )(x, w, b)


def kernel(r1_x, r1_e, r1_src, r1_dst, r1_gid, r2_x, r2_e, r2_src, r2_dst,
           r2_gid, pm_x, pm_e, pm_src, pm_dst, pm_gid, labels,
           pos_neg_sample, params):
    p = params
    f32 = _F32

    x = jnp.concatenate([r1_x, r2_x, pm_x], axis=0)
    src = jnp.concatenate(
        [r1_src, r2_src + N, pm_src + 2 * N]).astype(jnp.int32).reshape(NW, NCH, 128)
    dst = jnp.concatenate(
        [r1_dst, r2_dst + N, pm_dst + 2 * N]).astype(jnp.int32).reshape(16, NCS, 128)
    gidc = jnp.concatenate(
        [r1_gid, r2_gid + B, pm_gid + 2 * B]).astype(f32).reshape(NT, 1)

    w2 = p['edge_W'].reshape(DE * H, H)
    evt = jnp.concatenate([r1_e, r2_e, pm_e], axis=0).T
    bmat = p['edge_b'].reshape(H, H)
    cb = p['conv_b'].reshape(1, H)
    gws = ([p['gru_Wih'][k * H:(k + 1) * H].T for k in range(3)]
           + [p['gru_Whh'][k * H:(k + 1) * H].T for k in range(3)])
    gbs = ([p['gru_bih'][k * H:(k + 1) * H].reshape(1, H) for k in range(3)]
           + [p['gru_bhh'][k * H:(k + 1) * H].reshape(1, H) for k in range(3)])

    f = _proj(x, p['proj_W'], p['proj_b'].reshape(1, H))
    for _ in range(3):
        hs = _sc_gather(f, src)
        m = _msg(hs, evt, w2, bmat)
        agg = _sc_scatter(m, dst)
        f = _gru(agg, f, cb, gws, gbs)

    feat = f

    lwq = p['lstm_Wih'][:, :2 * H].T
    lwr = p['lstm_Wih'][:, 2 * H:].T
    lwh = p['lstm_Whh'].T
    lbi = p['lstm_bih'].reshape(1, 8 * H)
    lbh = p['lstm_bhh'].reshape(1, 8 * H)
    q = jnp.zeros((BT, 2 * H), f32)
    run = jnp.zeros((BT, 2 * H), f32)
    den = jnp.ones((BT, 1), f32)
    hl = jnp.zeros((BT, 2 * H), f32)
    cl = jnp.zeros((BT, 2 * H), f32)
    for _ in range(3):
        hl, cl = _lstm(q, run, den, hl, cl, lwq, lwr, lwh, lbi, lbh)
        q = hl
        _, den, run = _ro(feat, gidc, q)

    g = _sp(q, run, den, p['sp_W'][:2 * H], p['sp_W'][2 * H:],
            p['sp_b'].reshape(1, RO), p['sp_a'].reshape(1, 1))
    ge = g.reshape(NG, B, RO).transpose(1, 0, 2).reshape(B, NG * RO)

    pn = jnp.full((B, 1), pos_neg_sample, f32)
    ei = NG * RO + NC + 1
    eip = 3328
    xin = jnp.concatenate(
        [labels, ge, pn, jnp.zeros((B, eip - ei), f32)], axis=1)
    ew0 = jnp.concatenate(
        [p['enc_W0'], jnp.zeros((eip - ei, PH), f32)], axis=0)
    z = _linear_prelu(xin, ew0, p['enc_b0'].reshape(1, PH), p['enc_a0'].reshape(1, 1))
    z = _linear_prelu(z, p['enc_W1'], p['enc_b1'].reshape(1, PH), p['enc_a1'].reshape(1, 1))
    z = _linear_prelu(z, p['enc_W2'], p['enc_b2'].reshape(1, PH), p['enc_a2'].reshape(1, 1))
    eps = jax.random.normal(jax.random.key(42), (B, LAT), f32)
    mu, log_var, latent = _enc3(z, p['enc_W3'], p['enc_b3'].reshape(1, 2 * LAT), eps)

    di = NG * RO + LAT + 1
    dip = 3328
    yin = jnp.concatenate(
        [latent, ge, pn, jnp.zeros((B, dip - di), f32)], axis=1)
    dw0 = jnp.concatenate(
        [p['dec_W0'], jnp.zeros((dip - di, PH), f32)], axis=0)
    y = _linear_prelu(yin, dw0, p['dec_b0'].reshape(1, PH), p['dec_a0'].reshape(1, 1))
    y = _linear_prelu(y, p['dec_W1'], p['dec_b1'].reshape(1, PH), p['dec_a1'].reshape(1, 1))
    y = _linear_prelu(y, p['dec_W2'], p['dec_b2'].reshape(1, PH), p['dec_a2'].reshape(1, 1))
    dw3 = jnp.pad(p['dec_W3'], ((0, 0), (0, 128 - NC)))
    db3 = jnp.pad(p['dec_b3'], (0, 128 - NC)).reshape(1, 128)
    y = _dec3(y, dw3, db3)[:, :NC]
    return (y, mu, log_var)


# speedup vs baseline: 4.7100x; 1.0145x over previous
import functools

import jax
import jax.numpy as jnp
from jax import lax
from jax.experimental import pallas as pl
from jax.experimental.pallas import tpu as pltpu
from jax.experimental.pallas import tpu_sc as plsc

H = 64
DN = 128
DE = 16
RO = 1024
PH = 512
LAT = 128
NC = 100
B = 256
N = 8192
E = 16384
NG = 3
NT = NG * N
ET = NG * E
BT = NG * B

NW = 32
EPW = ET // NW
NCH = EPW // 128
RPS = NT // 16
HH = H // 4
EPS = ET // 16
NCS = EPS // 128

NB = 2048
EB = 1024
RB = 1024

_F32 = jnp.float32


def _dot(a, b):
    return jnp.dot(a, b, preferred_element_type=_F32)


def _sc_gather_body(f_hbm, idx_hbm, out_hbm, idx_v, rows_v, sem):
    wid = lax.axis_index("s") * 2 + lax.axis_index("c")
    pltpu.sync_copy(idx_hbm.at[wid], idx_v)
    for r in range(2):
        cps = []
        for j in range(NCH // 2):
            cps.append(
                pltpu.async_copy(f_hbm.at[idx_v.at[r * (NCH // 2) + j]],
                                 rows_v.at[pl.ds(j * 128, 128)], sem)
            )
        for cp in cps:
            cp.wait()
        pltpu.sync_copy(
            rows_v, out_hbm.at[pl.ds(wid * EPW + r * (EPW // 2), EPW // 2)])


def _sc_scatter_body(m_hbm, idx_hbm, out_hbm, idx_v, rows_v, zbuf, acc, sem, sem2):
    c = lax.axis_index("c")
    s = lax.axis_index("s")

    def _zrow(i, carry):
        zbuf[i, pl.ds(0, 16)] = jnp.zeros((16,), _F32)
        return carry

    lax.fori_loop(0, 128, _zrow, 0)
    pltpu.sync_copy(idx_hbm.at[s], idx_v)
    for p in range(2):
        cps = [pltpu.async_copy(zbuf, acc.at[pl.ds(s * RPS + j * 128, 128)], sem)
               for j in range(RPS // 128)]
        cps.append(pltpu.async_copy(
            m_hbm.at[pl.ds(s * EPS, EPS), pl.ds((2 * p + c) * HH, HH)], rows_v,
            sem2))
        for cp in cps:
            cp.wait()
        plsc.subcore_barrier()
        cps = [pltpu.async_copy(rows_v.at[pl.ds(j * 128, 128)], acc.at[idx_v.at[j]],
                                sem, add=True)
               for j in range(NCS)]
        for cp in cps:
            cp.wait()
        plsc.subcore_barrier()
        pltpu.sync_copy(acc.at[pl.ds(s * RPS, RPS)],
                        out_hbm.at[pl.ds(s * RPS, RPS), pl.ds((2 * p + c) * HH, HH)])


@functools.lru_cache(maxsize=None)
def _sc_kernels():
    mesh = plsc.VectorSubcoreMesh(core_axis_name="c", subcore_axis_name="s")
    gather = pl.kernel(
        _sc_gather_body,
        out_type=jax.ShapeDtypeStruct((ET, 2 * H), _F32),
        mesh=mesh,
        scratch_types=[
            pltpu.VMEM((NCH, 128), jnp.int32),
            pltpu.VMEM((EPW // 2, 2 * H), _F32),
            pltpu.SemaphoreType.DMA,
        ],
    )
    scatter = pl.kernel(
        _sc_scatter_body,
        out_type=jax.ShapeDtypeStruct((NT, H), _F32),
        mesh=mesh,
        compiler_params=pltpu.CompilerParams(use_tc_tiling_on_sc=False),
        scratch_types=[
            pltpu.VMEM((NCS, 128), jnp.int32),
            pltpu.VMEM((EPS, HH), _F32),
            pltpu.VMEM((128, HH), _F32),
            pltpu.VMEM_SHARED((NT, HH), _F32),
            pltpu.SemaphoreType.DMA,
            pltpu.SemaphoreType.DMA,
        ],
    )
    return gather, scatter


def _sc_gather(h, src):
    return _sc_kernels()[0](h, src)


def _sc_scatter(m, dst):
    return _sc_kernels()[1](m, dst)


def _proj_body(x_ref, w_ref, b_ref, o_ref):
    y = jnp.maximum(_dot(x_ref[...], w_ref[...]) + b_ref[...], 0.0)
    o_ref[...] = jnp.concatenate([y, y], axis=1)


def _proj(x, w, b):
    return pl.pallas_call(
        _proj_body,
        grid=(NT // NB,),
        in_specs=[
            pl.BlockSpec((NB, DN), lambda i: (i, 0)),
            pl.BlockSpec((DN, H), lambda i: (0, 0)),
            pl.BlockSpec((1, H), lambda i: (0, 0)),
        ],
        out_specs=pl.BlockSpec((NB, 2 * H), lambda i: (i, 0)),
        out_shape=jax.ShapeDtypeStruct((NT, 2 * H), _F32),
    )(x, w, b)


def _msg_body(hs_ref, evt_ref, w2_ref, bm_ref, o_ref):
    hst = jnp.transpose(hs_ref[...])[H:, :]
    evt = evt_ref[...]
    kt = (evt[:, None, :] * hst[None, :, :]).reshape(DE * H, EB)
    dn = (((0,), (0,)), ((), ()))
    o_ref[...] = (lax.dot_general(kt, w2_ref[...], dn, preferred_element_type=_F32)
                  + lax.dot_general(hst, bm_ref[...], dn, preferred_element_type=_F32))


def _msg(hs, evt, w2, bmat):
    return pl.pallas_call(
        _msg_body,
        grid=(ET // EB,),
        in_specs=[
            pl.BlockSpec((EB, 2 * H), lambda i: (i, 0)),
            pl.BlockSpec((DE, EB), lambda i: (0, i)),
            pl.BlockSpec((DE * H, H), lambda i: (0, 0)),
            pl.BlockSpec((H, H), lambda i: (0, 0)),
        ],
        out_specs=pl.BlockSpec((EB, H), lambda i: (i, 0)),
        out_shape=jax.ShapeDtypeStruct((ET, H), _F32),
    )(hs, evt, w2, bmat)


def _gru_body(ag_ref, f_ref, cb_ref, wir_ref, wiz_ref, win_ref,
              whr_ref, whz_ref, whn_ref, bir_ref, biz_ref, bin_ref,
              bhr_ref, bhz_ref, bhn_ref, o_ref):
    a = jnp.maximum(ag_ref[...] + cb_ref[...], 0.0)
    f = f_ref[...]
    h = f[:, H:]
    r = jax.nn.sigmoid(_dot(a, wir_ref[...]) + bir_ref[...]
                       + _dot(h, whr_ref[...]) + bhr_ref[...])
    z = jax.nn.sigmoid(_dot(a, wiz_ref[...]) + biz_ref[...]
                       + _dot(h, whz_ref[...]) + bhz_ref[...])
    n = jnp.tanh(_dot(a, win_ref[...]) + bin_ref[...]
                 + r * (_dot(h, whn_ref[...]) + bhn_ref[...]))
    o_ref[...] = jnp.concatenate([f[:, :H], (1.0 - z) * n + z * h], axis=1)


def _gru(ag, f, cb, ws, bs):
    mat = pl.BlockSpec((H, H), lambda i: (0, 0))
    vec = pl.BlockSpec((1, H), lambda i: (0, 0))
    big = pl.BlockSpec((NB, H), lambda i: (i, 0))
    wide = pl.BlockSpec((NB, 2 * H), lambda i: (i, 0))
    return pl.pallas_call(
        _gru_body,
        grid=(NT // NB,),
        in_specs=[big, wide, vec] + [mat] * 6 + [vec] * 6,
        out_specs=wide,
        out_shape=jax.ShapeDtypeStruct((NT, 2 * H), _F32),
    )(ag, f, cb, *ws, *bs)


def _ro_body(feat_ref, gid_ref, q_ref, mx_ref, den_ref, run_ref):
    i = pl.program_id(0)
    feat = feat_ref[...]
    gid = gid_ref[...]
    segs = lax.broadcasted_iota(jnp.int32, (RB, BT), 1).astype(_F32)
    mask = gid == segs
    qg = _dot(mask.astype(_F32), q_ref[...])
    es = jnp.sum(feat * qg, axis=1, keepdims=True)

    @pl.when(i == 0)
    def _():
        mx_ref[...] = jnp.full((1, BT), -1e30, _F32)
        den_ref[...] = jnp.zeros((BT, 1), _F32)
        run_ref[...] = jnp.zeros((BT, 2 * H), _F32)

    part = jnp.max(jnp.where(mask, es, -1e30), axis=0, keepdims=True)
    newmx = jnp.maximum(mx_ref[...], part)
    scale = jnp.transpose(jnp.exp(mx_ref[...] - newmx))
    mx_ref[...] = newmx
    emaxg = _dot(mask.astype(_F32), jnp.transpose(newmx))
    ex = jnp.exp(es - emaxg)
    exw = jnp.where(mask, ex, 0.0)
    dn = (((0,), (0,)), ((), ()))
    den_ref[...] = den_ref[...] * scale + lax.dot_general(
        exw, jnp.ones((RB, 1), _F32), dn, preferred_element_type=_F32)
    run_ref[...] = run_ref[...] * scale + lax.dot_general(
        exw, feat, dn, preferred_element_type=_F32)


def _ro(feat, gidc, q):
    return pl.pallas_call(
        _ro_body,
        grid=(NT // RB,),
        in_specs=[
            pl.BlockSpec((RB, 2 * H), lambda i: (i, 0)),
            pl.BlockSpec((RB, 1), lambda i: (i, 0)),
            pl.BlockSpec((BT, 2 * H), lambda i: (0, 0)),
        ],
        out_specs=[
            pl.BlockSpec((1, BT), lambda i: (0, 0)),
            pl.BlockSpec((BT, 1), lambda i: (0, 0)),
            pl.BlockSpec((BT, 2 * H), lambda i: (0, 0)),
        ],
        out_shape=[
            jax.ShapeDtypeStruct((1, BT), _F32),
            jax.ShapeDtypeStruct((BT, 1), _F32),
            jax.ShapeDtypeStruct((BT, 2 * H), _F32),
        ],
    )(feat, gidc, q)


def _lstm_body(q_ref, run_ref, den_ref, hl_ref, cl_ref, wq_ref, wr_ref,
               wh_ref, bi_ref, bh_ref, ho_ref, co_ref):
    r = run_ref[...] / jnp.maximum(den_ref[...], 1e-30)
    g = (_dot(q_ref[...], wq_ref[...]) + _dot(r, wr_ref[...])
         + _dot(hl_ref[...], wh_ref[...]) + bi_ref[...] + bh_ref[...])
    gi = jax.nn.sigmoid(g[:, 0 * LAT:1 * LAT])
    gf = jax.nn.sigmoid(g[:, 1 * LAT:2 * LAT])
    gg = jnp.tanh(g[:, 2 * LAT:3 * LAT])
    go = jax.nn.sigmoid(g[:, 3 * LAT:4 * LAT])
    c2 = gf * cl_ref[...] + gi * gg
    ho_ref[...] = go * jnp.tanh(c2)
    co_ref[...] = c2


def _lstm(q, run, den, hl, cl, wq, wr, wh, bi, bh):
    return pl.pallas_call(
        _lstm_body,
        out_shape=[
            jax.ShapeDtypeStruct((BT, 2 * H), _F32),
            jax.ShapeDtypeStruct((BT, 2 * H), _F32),
        ],
    )(q, run, den, hl, cl, wq, wr, wh, bi, bh)


def _sp_body(q_ref, run_ref, den_ref, wq_ref, wr_ref, b_ref, a_ref, o_ref):
    r = run_ref[...] / jnp.maximum(den_ref[...], 1e-30)
    g = _dot(q_ref[...], wq_ref[...]) + _dot(r, wr_ref[...]) + b_ref[...]
    a = a_ref[0, 0]
    o_ref[...] = jnp.where(g >= 0, g, a * g)


def _sp(q, run, den, wq, wr, b, a):
    return pl.pallas_call(
        _sp_body,
        out_shape=jax.ShapeDtypeStruct((BT, RO), _F32),
    )(q, run, den, wq, wr, b, a)


def _lin_body(x_ref, w_ref, b_ref, a_ref, o_ref):
    y = _dot(x_ref[...], w_ref[...]) + b_ref[...]
    a = a_ref[0, 0]
    o_ref[...] = jnp.where(y >= 0, y, a * y)


def _linear_prelu(x, w, b, a):
    return pl.pallas_call(
        _lin_body,
        out_shape=jax.ShapeDtypeStruct((x.shape[0], w.shape[1]), _F32),
    )(x, w, b, a)


def _enc3_body(x_ref, w_ref, b_ref, eps_ref, mu_ref, lv_ref, lat_ref):
    y = _dot(x_ref[...], w_ref[...]) + b_ref[...]
    mu = jnp.clip(y[:, :LAT], -10.0, 10.0)
    lv = jnp.clip(y[:, LAT:], -10.0, 10.0)
    mu_ref[...] = mu
    lv_ref[...] = lv
    lat_ref[...] = mu + eps_ref[...] * jnp.exp(0.5 * lv)


def _enc3(x, w, b, eps):
    return pl.pallas_call(
        _enc3_body,
        out_shape=[
            jax.ShapeDtypeStruct((B, LAT), _F32),
            jax.ShapeDtypeStruct((B, LAT), _F32),
            jax.ShapeDtypeStruct((B, LAT), _F32),
        ],
    )(x, w, b, eps)


def _dec3_body(x_ref, w_ref, b_ref, o_ref):
    o_ref[...] = jnp.clip(_dot(x_ref[...], w_ref[...]) + b_ref[...], -10.0, 10.0)


def _dec3(x, w, b):
    return pl.pallas_call(
        _dec3_body,
        out_shape=jax.ShapeDtypeStruct((B, 128), _F32),
    )(x, w, b)


def kernel(r1_x, r1_e, r1_src, r1_dst, r1_gid, r2_x, r2_e, r2_src, r2_dst,
           r2_gid, pm_x, pm_e, pm_src, pm_dst, pm_gid, labels,
           pos_neg_sample, params):
    p = params
    f32 = _F32

    x = jnp.concatenate([r1_x, r2_x, pm_x], axis=0)
    src = jnp.concatenate(
        [r1_src, r2_src + N, pm_src + 2 * N]).astype(jnp.int32).reshape(NW, NCH, 128)
    dst = jnp.concatenate(
        [r1_dst, r2_dst + N, pm_dst + 2 * N]).astype(jnp.int32).reshape(16, NCS, 128)
    gidc = jnp.concatenate(
        [r1_gid, r2_gid + B, pm_gid + 2 * B]).astype(f32).reshape(NT, 1)

    w2 = p['edge_W'].reshape(DE * H, H)
    evt = jnp.concatenate([r1_e, r2_e, pm_e], axis=0).T
    bmat = p['edge_b'].reshape(H, H)
    cb = p['conv_b'].reshape(1, H)
    gws = ([p['gru_Wih'][k * H:(k + 1) * H].T for k in range(3)]
           + [p['gru_Whh'][k * H:(k + 1) * H].T for k in range(3)])
    gbs = ([p['gru_bih'][k * H:(k + 1) * H].reshape(1, H) for k in range(3)]
           + [p['gru_bhh'][k * H:(k + 1) * H].reshape(1, H) for k in range(3)])

    f = _proj(x, p['proj_W'], p['proj_b'].reshape(1, H))
    for _ in range(3):
        hs = _sc_gather(f, src)
        m = _msg(hs, evt, w2, bmat)
        agg = _sc_scatter(m, dst)
        f = _gru(agg, f, cb, gws, gbs)

    feat = f

    lwq = p['lstm_Wih'][:, :2 * H].T
    lwr = p['lstm_Wih'][:, 2 * H:].T
    lwh = p['lstm_Whh'].T
    lbi = p['lstm_bih'].reshape(1, 8 * H)
    lbh = p['lstm_bhh'].reshape(1, 8 * H)
    q = jnp.zeros((BT, 2 * H), f32)
    run = jnp.zeros((BT, 2 * H), f32)
    den = jnp.ones((BT, 1), f32)
    hl = jnp.zeros((BT, 2 * H), f32)
    cl = jnp.zeros((BT, 2 * H), f32)
    for _ in range(3):
        hl, cl = _lstm(q, run, den, hl, cl, lwq, lwr, lwh, lbi, lbh)
        q = hl
        _, den, run = _ro(feat, gidc, q)

    g = _sp(q, run, den, p['sp_W'][:2 * H], p['sp_W'][2 * H:],
            p['sp_b'].reshape(1, RO), p['sp_a'].reshape(1, 1))
    ge = g.reshape(NG, B, RO).transpose(1, 0, 2).reshape(B, NG * RO)

    pn = jnp.full((B, 1), pos_neg_sample, f32)
    ei = NG * RO + NC + 1
    eip = 3328
    xin = jnp.concatenate(
        [labels, ge, pn, jnp.zeros((B, eip - ei), f32)], axis=1)
    ew0 = jnp.concatenate(
        [p['enc_W0'], jnp.zeros((eip - ei, PH), f32)], axis=0)
    z = _linear_prelu(xin, ew0, p['enc_b0'].reshape(1, PH), p['enc_a0'].reshape(1, 1))
    z = _linear_prelu(z, p['enc_W1'], p['enc_b1'].reshape(1, PH), p['enc_a1'].reshape(1, 1))
    z = _linear_prelu(z, p['enc_W2'], p['enc_b2'].reshape(1, PH), p['enc_a2'].reshape(1, 1))
    eps = jax.random.normal(jax.random.key(42), (B, LAT), f32)
    mu, log_var, latent = _enc3(z, p['enc_W3'], p['enc_b3'].reshape(1, 2 * LAT), eps)

    di = NG * RO + LAT + 1
    dip = 3328
    yin = jnp.concatenate(
        [latent, ge, pn, jnp.zeros((B, dip - di), f32)], axis=1)
    dw0 = jnp.concatenate(
        [p['dec_W0'], jnp.zeros((dip - di, PH), f32)], axis=0)
    y = _linear_prelu(yin, dw0, p['dec_b0'].reshape(1, PH), p['dec_a0'].reshape(1, 1))
    y = _linear_prelu(y, p['dec_W1'], p['dec_b1'].reshape(1, PH), p['dec_a1'].reshape(1, 1))
    y = _linear_prelu(y, p['dec_W2'], p['dec_b2'].reshape(1, PH), p['dec_a2'].reshape(1, 1))
    dw3 = jnp.pad(p['dec_W3'], ((0, 0), (0, 128 - NC)))
    db3 = jnp.pad(p['dec_b3'], (0, 128 - NC)).reshape(1, 128)
    y = _dec3(y, dw3, db3)[:, :NC]
    return (y, mu, log_var)


# speedup vs baseline: 4.8292x; 1.0253x over previous
import functools

import jax
import jax.numpy as jnp
from jax import lax
from jax.experimental import pallas as pl
from jax.experimental.pallas import tpu as pltpu
from jax.experimental.pallas import tpu_sc as plsc

H = 64
DN = 128
DE = 16
RO = 1024
PH = 512
LAT = 128
NC = 100
B = 256
N = 8192
E = 16384
NG = 3
NT = NG * N
ET = NG * E
BT = NG * B

NW = 32
EPW = ET // NW
NCH = EPW // 128
RPS = NT // 16
HH = H // 4
EPS = ET // 16
NCS = EPS // 128

NB = 2048
EB = 1024
RB = 1024

_F32 = jnp.float32


def _dot(a, b):
    return jnp.dot(a, b, preferred_element_type=_F32)


def _sc_gather_body(f_hbm, idx_hbm, out_hbm, idx_v, rows_v, sem):
    wid = lax.axis_index("s") * 2 + lax.axis_index("c")
    pltpu.sync_copy(idx_hbm.at[wid], idx_v)
    for r in range(2):
        cps = []
        for j in range(NCH // 2):
            cps.append(
                pltpu.async_copy(f_hbm.at[idx_v.at[r * (NCH // 2) + j]],
                                 rows_v.at[pl.ds(j * 128, 128)], sem)
            )
        for cp in cps:
            cp.wait()
        pltpu.sync_copy(
            rows_v, out_hbm.at[pl.ds(wid * EPW + r * (EPW // 2), EPW // 2)])


def _sc_scatter_body(m_hbm, idx_hbm, out_hbm, idx_v, rows_v, zbuf, acc, sem, sem2):
    c = lax.axis_index("c")
    s = lax.axis_index("s")

    def _zrow(i, carry):
        zbuf[i, pl.ds(0, 16)] = jnp.zeros((16,), _F32)
        return carry

    lax.fori_loop(0, 128, _zrow, 0)
    pltpu.sync_copy(idx_hbm.at[s], idx_v)
    for p in range(2):
        cps = [pltpu.async_copy(zbuf, acc.at[pl.ds(s * RPS + j * 128, 128)], sem)
               for j in range(RPS // 128)]
        cps.append(pltpu.async_copy(
            m_hbm.at[pl.ds(s * EPS, EPS), pl.ds((2 * p + c) * HH, HH)], rows_v,
            sem2))
        for cp in cps:
            cp.wait()
        plsc.subcore_barrier()
        cps = [pltpu.async_copy(rows_v.at[pl.ds(j * 128, 128)], acc.at[idx_v.at[j]],
                                sem, add=True)
               for j in range(NCS)]
        for cp in cps:
            cp.wait()
        plsc.subcore_barrier()
        pltpu.sync_copy(acc.at[pl.ds(s * RPS, RPS)],
                        out_hbm.at[pl.ds(s * RPS, RPS), pl.ds((2 * p + c) * HH, HH)])


@functools.lru_cache(maxsize=None)
def _sc_kernels():
    mesh = plsc.VectorSubcoreMesh(core_axis_name="c", subcore_axis_name="s")
    gather = pl.kernel(
        _sc_gather_body,
        out_type=jax.ShapeDtypeStruct((ET, 2 * H), _F32),
        mesh=mesh,
        scratch_types=[
            pltpu.VMEM((NCH, 128), jnp.int32),
            pltpu.VMEM((EPW // 2, 2 * H), _F32),
            pltpu.SemaphoreType.DMA,
        ],
    )
    scatter = pl.kernel(
        _sc_scatter_body,
        out_type=jax.ShapeDtypeStruct((NT, H), _F32),
        mesh=mesh,
        compiler_params=pltpu.CompilerParams(use_tc_tiling_on_sc=False),
        scratch_types=[
            pltpu.VMEM((NCS, 128), jnp.int32),
            pltpu.VMEM((EPS, HH), _F32),
            pltpu.VMEM((128, HH), _F32),
            pltpu.VMEM_SHARED((NT, HH), _F32),
            pltpu.SemaphoreType.DMA,
            pltpu.SemaphoreType.DMA,
        ],
    )
    return gather, scatter


def _sc_gather(h, src):
    return _sc_kernels()[0](h, src)


def _sc_scatter(m, dst):
    return _sc_kernels()[1](m, dst)


def _proj_body(x_ref, w_ref, b_ref, o_ref):
    y = jnp.maximum(_dot(x_ref[...], w_ref[...]) + b_ref[...], 0.0)
    o_ref[...] = jnp.concatenate([y, y], axis=1)


def _proj(x, w, b):
    return pl.pallas_call(
        _proj_body,
        grid=(NT // NB,),
        in_specs=[
            pl.BlockSpec((NB, DN), lambda i: (i, 0)),
            pl.BlockSpec((DN, H), lambda i: (0, 0)),
            pl.BlockSpec((1, H), lambda i: (0, 0)),
        ],
        out_specs=pl.BlockSpec((NB, 2 * H), lambda i: (i, 0)),
        out_shape=jax.ShapeDtypeStruct((NT, 2 * H), _F32),
    )(x, w, b)


def _msg_body(hs_ref, evt_ref, w2_ref, bm_ref, o_ref):
    hst = jnp.transpose(hs_ref[...])[H:, :]
    evt = evt_ref[...]
    kt = (evt[:, None, :] * hst[None, :, :]).reshape(DE * H, EB)
    dn = (((0,), (0,)), ((), ()))
    o_ref[...] = (lax.dot_general(kt, w2_ref[...], dn, preferred_element_type=_F32)
                  + lax.dot_general(hst, bm_ref[...], dn, preferred_element_type=_F32))


def _msg(hs, evt, w2, bmat):
    return pl.pallas_call(
        _msg_body,
        grid=(ET // EB,),
        in_specs=[
            pl.BlockSpec((EB, 2 * H), lambda i: (i, 0)),
            pl.BlockSpec((DE, EB), lambda i: (0, i)),
            pl.BlockSpec((DE * H, H), lambda i: (0, 0)),
            pl.BlockSpec((H, H), lambda i: (0, 0)),
        ],
        out_specs=pl.BlockSpec((EB, H), lambda i: (i, 0)),
        out_shape=jax.ShapeDtypeStruct((ET, H), _F32),
    )(hs, evt, w2, bmat)


def _gru_body(ag_ref, f_ref, cb_ref, wir_ref, wiz_ref, win_ref,
              whr_ref, whz_ref, whn_ref, bir_ref, biz_ref, bin_ref,
              bhr_ref, bhz_ref, bhn_ref, o_ref):
    a = jnp.maximum(ag_ref[...] + cb_ref[...], 0.0)
    f = f_ref[...]
    h = f[:, H:]
    r = jax.nn.sigmoid(_dot(a, wir_ref[...]) + bir_ref[...]
                       + _dot(h, whr_ref[...]) + bhr_ref[...])
    z = jax.nn.sigmoid(_dot(a, wiz_ref[...]) + biz_ref[...]
                       + _dot(h, whz_ref[...]) + bhz_ref[...])
    n = jnp.tanh(_dot(a, win_ref[...]) + bin_ref[...]
                 + r * (_dot(h, whn_ref[...]) + bhn_ref[...]))
    o_ref[...] = jnp.concatenate([f[:, :H], (1.0 - z) * n + z * h], axis=1)


def _gru(ag, f, cb, ws, bs):
    mat = pl.BlockSpec((H, H), lambda i: (0, 0))
    vec = pl.BlockSpec((1, H), lambda i: (0, 0))
    big = pl.BlockSpec((NB, H), lambda i: (i, 0))
    wide = pl.BlockSpec((NB, 2 * H), lambda i: (i, 0))
    return pl.pallas_call(
        _gru_body,
        grid=(NT // NB,),
        in_specs=[big, wide, vec] + [mat] * 6 + [vec] * 6,
        out_specs=wide,
        out_shape=jax.ShapeDtypeStruct((NT, 2 * H), _F32),
    )(ag, f, cb, *ws, *bs)


def _ro_body(feat_ref, gid_ref, q_ref, mx_ref, den_ref, run_ref):
    i = pl.program_id(0)
    feat = feat_ref[...]
    gid = gid_ref[...]
    segs = lax.broadcasted_iota(jnp.int32, (RB, BT), 1).astype(_F32)
    mask = gid == segs
    qg = _dot(mask.astype(_F32), q_ref[...])
    es = jnp.sum(feat * qg, axis=1, keepdims=True)

    @pl.when(i == 0)
    def _():
        mx_ref[...] = jnp.full((1, BT), -1e30, _F32)
        den_ref[...] = jnp.zeros((BT, 1), _F32)
        run_ref[...] = jnp.zeros((BT, 2 * H), _F32)

    part = jnp.max(jnp.where(mask, es, -1e30), axis=0, keepdims=True)
    newmx = jnp.maximum(mx_ref[...], part)
    scale = jnp.transpose(jnp.exp(mx_ref[...] - newmx))
    mx_ref[...] = newmx
    emaxg = _dot(mask.astype(_F32), jnp.transpose(newmx))
    ex = jnp.exp(es - emaxg)
    exw = jnp.where(mask, ex, 0.0)
    dn = (((0,), (0,)), ((), ()))
    den_ref[...] = den_ref[...] * scale + lax.dot_general(
        exw, jnp.ones((RB, 1), _F32), dn, preferred_element_type=_F32)
    run_ref[...] = run_ref[...] * scale + lax.dot_general(
        exw, feat, dn, preferred_element_type=_F32)


def _ro(feat, gidc, q):
    return pl.pallas_call(
        _ro_body,
        grid=(NT // RB,),
        in_specs=[
            pl.BlockSpec((RB, 2 * H), lambda i: (i, 0)),
            pl.BlockSpec((RB, 1), lambda i: (i, 0)),
            pl.BlockSpec((BT, 2 * H), lambda i: (0, 0)),
        ],
        out_specs=[
            pl.BlockSpec((1, BT), lambda i: (0, 0)),
            pl.BlockSpec((BT, 1), lambda i: (0, 0)),
            pl.BlockSpec((BT, 2 * H), lambda i: (0, 0)),
        ],
        out_shape=[
            jax.ShapeDtypeStruct((1, BT), _F32),
            jax.ShapeDtypeStruct((BT, 1), _F32),
            jax.ShapeDtypeStruct((BT, 2 * H), _F32),
        ],
    )(feat, gidc, q)


def _lstm_body(q_ref, run_ref, den_ref, hl_ref, cl_ref, wq_ref, wr_ref,
               wh_ref, bi_ref, bh_ref, ho_ref, co_ref):
    r = run_ref[...] / jnp.maximum(den_ref[...], 1e-30)
    g = (_dot(q_ref[...], wq_ref[...]) + _dot(r, wr_ref[...])
         + _dot(hl_ref[...], wh_ref[...]) + bi_ref[...] + bh_ref[...])
    gi = jax.nn.sigmoid(g[:, 0 * LAT:1 * LAT])
    gf = jax.nn.sigmoid(g[:, 1 * LAT:2 * LAT])
    gg = jnp.tanh(g[:, 2 * LAT:3 * LAT])
    go = jax.nn.sigmoid(g[:, 3 * LAT:4 * LAT])
    c2 = gf * cl_ref[...] + gi * gg
    ho_ref[...] = go * jnp.tanh(c2)
    co_ref[...] = c2


def _lstm(q, run, den, hl, cl, wq, wr, wh, bi, bh):
    return pl.pallas_call(
        _lstm_body,
        out_shape=[
            jax.ShapeDtypeStruct((BT, 2 * H), _F32),
            jax.ShapeDtypeStruct((BT, 2 * H), _F32),
        ],
    )(q, run, den, hl, cl, wq, wr, wh, bi, bh)


def _sp_body(q_ref, run_ref, den_ref, wq_ref, wr_ref, b_ref, a_ref, o_ref):
    r = run_ref[...] / jnp.maximum(den_ref[...], 1e-30)
    g = _dot(q_ref[...], wq_ref[...]) + _dot(r, wr_ref[...]) + b_ref[...]
    a = a_ref[0, 0]
    o_ref[...] = jnp.where(g >= 0, g, a * g)


def _sp(q, run, den, wq, wr, b, a):
    return pl.pallas_call(
        _sp_body,
        out_shape=jax.ShapeDtypeStruct((BT, RO), _F32),
    )(q, run, den, wq, wr, b, a)


def _prelu(y, a_ref):
    a = a_ref[0, 0]
    return jnp.where(y >= 0, y, a * y)


def _vae_body(labp_ref, g_ref, eps_ref,
              wlab_ref, wge_ref, beff_ref, a0_ref,
              ew1_ref, eb1_ref, a1_ref, ew2_ref, eb2_ref, a2_ref,
              ew3_ref, eb3_ref,
              dwlat_ref, dwge_ref, dbeff_ref, d0_ref,
              dw1_ref, db1_ref, d1_ref, dw2_ref, db2_ref, d2_ref,
              dw3_ref, db3_ref,
              y_ref, mu_ref, lv_ref):
    g = g_ref[...]

    def ge_mm(w_ref):
        acc = _dot(g[0 * B:1 * B, :], w_ref[0 * RO:1 * RO, :])
        acc += _dot(g[1 * B:2 * B, :], w_ref[1 * RO:2 * RO, :])
        acc += _dot(g[2 * B:3 * B, :], w_ref[2 * RO:3 * RO, :])
        return acc

    x = _prelu(_dot(labp_ref[...], wlab_ref[...]) + ge_mm(wge_ref)
               + beff_ref[...], a0_ref)
    x = _prelu(_dot(x, ew1_ref[...]) + eb1_ref[...], a1_ref)
    x = _prelu(_dot(x, ew2_ref[...]) + eb2_ref[...], a2_ref)
    y = _dot(x, ew3_ref[...]) + eb3_ref[...]
    mu = jnp.clip(y[:, :LAT], -10.0, 10.0)
    lv = jnp.clip(y[:, LAT:], -10.0, 10.0)
    mu_ref[...] = mu
    lv_ref[...] = lv
    lat = mu + eps_ref[...] * jnp.exp(0.5 * lv)
    z = _prelu(_dot(lat, dwlat_ref[...]) + ge_mm(dwge_ref)
               + dbeff_ref[...], d0_ref)
    z = _prelu(_dot(z, dw1_ref[...]) + db1_ref[...], d1_ref)
    z = _prelu(_dot(z, dw2_ref[...]) + db2_ref[...], d2_ref)
    y_ref[...] = jnp.clip(_dot(z, dw3_ref[...]) + db3_ref[...], -10.0, 10.0)


def _vae_mlp(labp, g, eps, ws):
    return pl.pallas_call(
        _vae_body,
        out_shape=[
            jax.ShapeDtypeStruct((B, 128), _F32),
            jax.ShapeDtypeStruct((B, LAT), _F32),
            jax.ShapeDtypeStruct((B, LAT), _F32),
        ],
    )(labp, g, eps, *ws)


def kernel(r1_x, r1_e, r1_src, r1_dst, r1_gid, r2_x, r2_e, r2_src, r2_dst,
           r2_gid, pm_x, pm_e, pm_src, pm_dst, pm_gid, labels,
           pos_neg_sample, params):
    p = params
    f32 = _F32

    x = jnp.concatenate([r1_x, r2_x, pm_x], axis=0)
    src = jnp.concatenate(
        [r1_src, r2_src + N, pm_src + 2 * N]).astype(jnp.int32).reshape(NW, NCH, 128)
    dst = jnp.concatenate(
        [r1_dst, r2_dst + N, pm_dst + 2 * N]).astype(jnp.int32).reshape(16, NCS, 128)
    gidc = jnp.concatenate(
        [r1_gid, r2_gid + B, pm_gid + 2 * B]).astype(f32).reshape(NT, 1)

    w2 = p['edge_W'].reshape(DE * H, H)
    evt = jnp.concatenate([r1_e, r2_e, pm_e], axis=0).T
    bmat = p['edge_b'].reshape(H, H)
    cb = p['conv_b'].reshape(1, H)
    gws = ([p['gru_Wih'][k * H:(k + 1) * H].T for k in range(3)]
           + [p['gru_Whh'][k * H:(k + 1) * H].T for k in range(3)])
    gbs = ([p['gru_bih'][k * H:(k + 1) * H].reshape(1, H) for k in range(3)]
           + [p['gru_bhh'][k * H:(k + 1) * H].reshape(1, H) for k in range(3)])

    f = _proj(x, p['proj_W'], p['proj_b'].reshape(1, H))
    for _ in range(3):
        hs = _sc_gather(f, src)
        m = _msg(hs, evt, w2, bmat)
        agg = _sc_scatter(m, dst)
        f = _gru(agg, f, cb, gws, gbs)

    feat = f

    lwq = p['lstm_Wih'][:, :2 * H].T
    lwr = p['lstm_Wih'][:, 2 * H:].T
    lwh = p['lstm_Whh'].T
    lbi = p['lstm_bih'].reshape(1, 8 * H)
    lbh = p['lstm_bhh'].reshape(1, 8 * H)
    q = jnp.zeros((BT, 2 * H), f32)
    run = jnp.zeros((BT, 2 * H), f32)
    den = jnp.ones((BT, 1), f32)
    hl = jnp.zeros((BT, 2 * H), f32)
    cl = jnp.zeros((BT, 2 * H), f32)
    for _ in range(3):
        hl, cl = _lstm(q, run, den, hl, cl, lwq, lwr, lwh, lbi, lbh)
        q = hl
        _, den, run = _ro(feat, gidc, q)

    g = _sp(q, run, den, p['sp_W'][:2 * H], p['sp_W'][2 * H:],
            p['sp_b'].reshape(1, RO), p['sp_a'].reshape(1, 1))

    posf = jnp.asarray(pos_neg_sample).astype(f32)
    labp = jnp.pad(labels, ((0, 0), (0, 128 - NC)))
    eps = jax.random.normal(jax.random.key(42), (B, LAT), f32)
    gei = NC + NG * RO
    ws = [
        jnp.pad(p['enc_W0'][:NC], ((0, 128 - NC), (0, 0))),
        p['enc_W0'][NC:gei],
        (p['enc_b0'] + posf * p['enc_W0'][gei]).reshape(1, PH),
        p['enc_a0'].reshape(1, 1),
        p['enc_W1'], p['enc_b1'].reshape(1, PH), p['enc_a1'].reshape(1, 1),
        p['enc_W2'], p['enc_b2'].reshape(1, PH), p['enc_a2'].reshape(1, 1),
        p['enc_W3'], p['enc_b3'].reshape(1, 2 * LAT),
        p['dec_W0'][:LAT],
        p['dec_W0'][LAT:LAT + NG * RO],
        (p['dec_b0'] + posf * p['dec_W0'][LAT + NG * RO]).reshape(1, PH),
        p['dec_a0'].reshape(1, 1),
        p['dec_W1'], p['dec_b1'].reshape(1, PH), p['dec_a1'].reshape(1, 1),
        p['dec_W2'], p['dec_b2'].reshape(1, PH), p['dec_a2'].reshape(1, 1),
        jnp.pad(p['dec_W3'], ((0, 0), (0, 128 - NC))),
        jnp.pad(p['dec_b3'], (0, 128 - NC)).reshape(1, 128),
    ]
    y, mu, log_var = _vae_mlp(labp, g, eps, ws)
    y = y[:, :NC]
    return (y, mu, log_var)


# speedup vs baseline: 5.0425x; 1.0442x over previous
import functools

import jax
import jax.numpy as jnp
from jax import lax
from jax.experimental import pallas as pl
from jax.experimental.pallas import tpu as pltpu
from jax.experimental.pallas import tpu_sc as plsc

H = 64
DN = 128
DE = 16
RO = 1024
PH = 512
LAT = 128
NC = 100
B = 256
N = 8192
E = 16384
NG = 3
NT = NG * N
ET = NG * E
BT = NG * B

NW = 32
EPW = ET // NW
NCH = EPW // 128
RPS = NT // 16
HH = H // 4
EPS = ET // 16
NCS = EPS // 128

NB = 2048
EB = 2048
RB = 2048

_F32 = jnp.float32


def _dot(a, b):
    return jnp.dot(a, b, preferred_element_type=_F32)


def _sc_gather_body(f_hbm, idx_hbm, out_hbm, idx_v, rows_v, sem):
    wid = lax.axis_index("s") * 2 + lax.axis_index("c")
    pltpu.sync_copy(idx_hbm.at[wid], idx_v)
    for r in range(2):
        cps = []
        for j in range(NCH // 2):
            cps.append(
                pltpu.async_copy(f_hbm.at[idx_v.at[r * (NCH // 2) + j]],
                                 rows_v.at[pl.ds(j * 128, 128)], sem)
            )
        for cp in cps:
            cp.wait()
        pltpu.sync_copy(
            rows_v, out_hbm.at[pl.ds(wid * EPW + r * (EPW // 2), EPW // 2)])


def _sc_scatter_body(m_hbm, idx_hbm, out_hbm, idx_v, rows_v, zbuf, acc, sem, sem2):
    c = lax.axis_index("c")
    s = lax.axis_index("s")

    def _zrow(i, carry):
        zbuf[i, pl.ds(0, 16)] = jnp.zeros((16,), _F32)
        return carry

    lax.fori_loop(0, 128, _zrow, 0)
    pltpu.sync_copy(idx_hbm.at[s], idx_v)
    for p in range(2):
        cps = [pltpu.async_copy(zbuf, acc.at[pl.ds(s * RPS + j * 128, 128)], sem)
               for j in range(RPS // 128)]
        cps.append(pltpu.async_copy(
            m_hbm.at[pl.ds(s * EPS, EPS), pl.ds((2 * p + c) * HH, HH)], rows_v,
            sem2))
        for cp in cps:
            cp.wait()
        plsc.subcore_barrier()
        cps = [pltpu.async_copy(rows_v.at[pl.ds(j * 128, 128)], acc.at[idx_v.at[j]],
                                sem, add=True)
               for j in range(NCS)]
        for cp in cps:
            cp.wait()
        plsc.subcore_barrier()
        pltpu.sync_copy(acc.at[pl.ds(s * RPS, RPS)],
                        out_hbm.at[pl.ds(s * RPS, RPS), pl.ds((2 * p + c) * HH, HH)])


@functools.lru_cache(maxsize=None)
def _sc_kernels():
    mesh = plsc.VectorSubcoreMesh(core_axis_name="c", subcore_axis_name="s")
    gather = pl.kernel(
        _sc_gather_body,
        out_type=jax.ShapeDtypeStruct((ET, 2 * H), _F32),
        mesh=mesh,
        scratch_types=[
            pltpu.VMEM((NCH, 128), jnp.int32),
            pltpu.VMEM((EPW // 2, 2 * H), _F32),
            pltpu.SemaphoreType.DMA,
        ],
    )
    scatter = pl.kernel(
        _sc_scatter_body,
        out_type=jax.ShapeDtypeStruct((NT, H), _F32),
        mesh=mesh,
        compiler_params=pltpu.CompilerParams(use_tc_tiling_on_sc=False),
        scratch_types=[
            pltpu.VMEM((NCS, 128), jnp.int32),
            pltpu.VMEM((EPS, HH), _F32),
            pltpu.VMEM((128, HH), _F32),
            pltpu.VMEM_SHARED((NT, HH), _F32),
            pltpu.SemaphoreType.DMA,
            pltpu.SemaphoreType.DMA,
        ],
    )
    return gather, scatter


def _sc_gather(h, src):
    return _sc_kernels()[0](h, src)


def _sc_scatter(m, dst):
    return _sc_kernels()[1](m, dst)


def _proj_body(x_ref, w_ref, b_ref, o_ref):
    y = jnp.maximum(_dot(x_ref[...], w_ref[...]) + b_ref[...], 0.0)
    o_ref[...] = jnp.concatenate([y, y], axis=1)


def _proj(x, w, b):
    return pl.pallas_call(
        _proj_body,
        grid=(NT // NB,),
        in_specs=[
            pl.BlockSpec((NB, DN), lambda i: (i, 0)),
            pl.BlockSpec((DN, H), lambda i: (0, 0)),
            pl.BlockSpec((1, H), lambda i: (0, 0)),
        ],
        out_specs=pl.BlockSpec((NB, 2 * H), lambda i: (i, 0)),
        out_shape=jax.ShapeDtypeStruct((NT, 2 * H), _F32),
    )(x, w, b)


def _msg_body(hs_ref, evt_ref, w2_ref, bm_ref, o_ref):
    hst = jnp.transpose(hs_ref[...])[H:, :]
    evt = evt_ref[...]
    kt = (evt[:, None, :] * hst[None, :, :]).reshape(DE * H, EB)
    dn = (((0,), (0,)), ((), ()))
    o_ref[...] = (lax.dot_general(kt, w2_ref[...], dn, preferred_element_type=_F32)
                  + lax.dot_general(hst, bm_ref[...], dn, preferred_element_type=_F32))


def _msg(hs, evt, w2, bmat):
    return pl.pallas_call(
        _msg_body,
        grid=(ET // EB,),
        in_specs=[
            pl.BlockSpec((EB, 2 * H), lambda i: (i, 0)),
            pl.BlockSpec((DE, EB), lambda i: (0, i)),
            pl.BlockSpec((DE * H, H), lambda i: (0, 0)),
            pl.BlockSpec((H, H), lambda i: (0, 0)),
        ],
        out_specs=pl.BlockSpec((EB, H), lambda i: (i, 0)),
        out_shape=jax.ShapeDtypeStruct((ET, H), _F32),
    )(hs, evt, w2, bmat)


def _gru_body(ag_ref, f_ref, cb_ref, wir_ref, wiz_ref, win_ref,
              whr_ref, whz_ref, whn_ref, bir_ref, biz_ref, bin_ref,
              bhr_ref, bhz_ref, bhn_ref, o_ref):
    a = jnp.maximum(ag_ref[...] + cb_ref[...], 0.0)
    f = f_ref[...]
    h = f[:, H:]
    r = jax.nn.sigmoid(_dot(a, wir_ref[...]) + bir_ref[...]
                       + _dot(h, whr_ref[...]) + bhr_ref[...])
    z = jax.nn.sigmoid(_dot(a, wiz_ref[...]) + biz_ref[...]
                       + _dot(h, whz_ref[...]) + bhz_ref[...])
    n = jnp.tanh(_dot(a, win_ref[...]) + bin_ref[...]
                 + r * (_dot(h, whn_ref[...]) + bhn_ref[...]))
    o_ref[...] = jnp.concatenate([f[:, :H], (1.0 - z) * n + z * h], axis=1)


def _gru(ag, f, cb, ws, bs):
    mat = pl.BlockSpec((H, H), lambda i: (0, 0))
    vec = pl.BlockSpec((1, H), lambda i: (0, 0))
    big = pl.BlockSpec((NB, H), lambda i: (i, 0))
    wide = pl.BlockSpec((NB, 2 * H), lambda i: (i, 0))
    return pl.pallas_call(
        _gru_body,
        grid=(NT // NB,),
        in_specs=[big, wide, vec] + [mat] * 6 + [vec] * 6,
        out_specs=wide,
        out_shape=jax.ShapeDtypeStruct((NT, 2 * H), _F32),
    )(ag, f, cb, *ws, *bs)


def _ro_body(feat_ref, gid_ref, q_ref, mx_ref, den_ref, run_ref):
    i = pl.program_id(0)
    feat = feat_ref[...]
    gid = gid_ref[...]
    segs = lax.broadcasted_iota(jnp.int32, (RB, BT), 1).astype(_F32)
    mask = gid == segs
    qg = _dot(mask.astype(_F32), q_ref[...])
    es = jnp.sum(feat * qg, axis=1, keepdims=True)

    @pl.when(i == 0)
    def _():
        mx_ref[...] = jnp.full((1, BT), -1e30, _F32)
        den_ref[...] = jnp.zeros((BT, 1), _F32)
        run_ref[...] = jnp.zeros((BT, 2 * H), _F32)

    part = jnp.max(jnp.where(mask, es, -1e30), axis=0, keepdims=True)
    newmx = jnp.maximum(mx_ref[...], part)
    scale = jnp.transpose(jnp.exp(mx_ref[...] - newmx))
    mx_ref[...] = newmx
    emaxg = _dot(mask.astype(_F32), jnp.transpose(newmx))
    ex = jnp.exp(es - emaxg)
    exw = jnp.where(mask, ex, 0.0)
    dn = (((0,), (0,)), ((), ()))
    den_ref[...] = den_ref[...] * scale + lax.dot_general(
        exw, jnp.ones((RB, 1), _F32), dn, preferred_element_type=_F32)
    run_ref[...] = run_ref[...] * scale + lax.dot_general(
        exw, feat, dn, preferred_element_type=_F32)


def _ro(feat, gidc, q):
    return pl.pallas_call(
        _ro_body,
        grid=(NT // RB,),
        in_specs=[
            pl.BlockSpec((RB, 2 * H), lambda i: (i, 0)),
            pl.BlockSpec((RB, 1), lambda i: (i, 0)),
            pl.BlockSpec((BT, 2 * H), lambda i: (0, 0)),
        ],
        out_specs=[
            pl.BlockSpec((1, BT), lambda i: (0, 0)),
            pl.BlockSpec((BT, 1), lambda i: (0, 0)),
            pl.BlockSpec((BT, 2 * H), lambda i: (0, 0)),
        ],
        out_shape=[
            jax.ShapeDtypeStruct((1, BT), _F32),
            jax.ShapeDtypeStruct((BT, 1), _F32),
            jax.ShapeDtypeStruct((BT, 2 * H), _F32),
        ],
    )(feat, gidc, q)


def _lstm_body(q_ref, run_ref, den_ref, hl_ref, cl_ref, wq_ref, wr_ref,
               wh_ref, bi_ref, bh_ref, ho_ref, co_ref):
    r = run_ref[...] / jnp.maximum(den_ref[...], 1e-30)
    g = (_dot(q_ref[...], wq_ref[...]) + _dot(r, wr_ref[...])
         + _dot(hl_ref[...], wh_ref[...]) + bi_ref[...] + bh_ref[...])
    gi = jax.nn.sigmoid(g[:, 0 * LAT:1 * LAT])
    gf = jax.nn.sigmoid(g[:, 1 * LAT:2 * LAT])
    gg = jnp.tanh(g[:, 2 * LAT:3 * LAT])
    go = jax.nn.sigmoid(g[:, 3 * LAT:4 * LAT])
    c2 = gf * cl_ref[...] + gi * gg
    ho_ref[...] = go * jnp.tanh(c2)
    co_ref[...] = c2


def _lstm(q, run, den, hl, cl, wq, wr, wh, bi, bh):
    return pl.pallas_call(
        _lstm_body,
        out_shape=[
            jax.ShapeDtypeStruct((BT, 2 * H), _F32),
            jax.ShapeDtypeStruct((BT, 2 * H), _F32),
        ],
    )(q, run, den, hl, cl, wq, wr, wh, bi, bh)


def _sp_body(q_ref, run_ref, den_ref, wq_ref, wr_ref, b_ref, a_ref, o_ref):
    r = run_ref[...] / jnp.maximum(den_ref[...], 1e-30)
    g = _dot(q_ref[...], wq_ref[...]) + _dot(r, wr_ref[...]) + b_ref[...]
    a = a_ref[0, 0]
    o_ref[...] = jnp.where(g >= 0, g, a * g)


def _sp(q, run, den, wq, wr, b, a):
    return pl.pallas_call(
        _sp_body,
        out_shape=jax.ShapeDtypeStruct((BT, RO), _F32),
    )(q, run, den, wq, wr, b, a)


def _prelu(y, a_ref):
    a = a_ref[0, 0]
    return jnp.where(y >= 0, y, a * y)


def _vae_body(labp_ref, g_ref, eps_ref,
              wlab_ref, wge_ref, beff_ref, a0_ref,
              ew1_ref, eb1_ref, a1_ref, ew2_ref, eb2_ref, a2_ref,
              ew3_ref, eb3_ref,
              dwlat_ref, dwge_ref, dbeff_ref, d0_ref,
              dw1_ref, db1_ref, d1_ref, dw2_ref, db2_ref, d2_ref,
              dw3_ref, db3_ref,
              y_ref, mu_ref, lv_ref):
    g = g_ref[...]

    def ge_mm(w_ref):
        acc = _dot(g[0 * B:1 * B, :], w_ref[0 * RO:1 * RO, :])
        acc += _dot(g[1 * B:2 * B, :], w_ref[1 * RO:2 * RO, :])
        acc += _dot(g[2 * B:3 * B, :], w_ref[2 * RO:3 * RO, :])
        return acc

    x = _prelu(_dot(labp_ref[...], wlab_ref[...]) + ge_mm(wge_ref)
               + beff_ref[...], a0_ref)
    x = _prelu(_dot(x, ew1_ref[...]) + eb1_ref[...], a1_ref)
    x = _prelu(_dot(x, ew2_ref[...]) + eb2_ref[...], a2_ref)
    y = _dot(x, ew3_ref[...]) + eb3_ref[...]
    mu = jnp.clip(y[:, :LAT], -10.0, 10.0)
    lv = jnp.clip(y[:, LAT:], -10.0, 10.0)
    mu_ref[...] = mu
    lv_ref[...] = lv
    lat = mu + eps_ref[...] * jnp.exp(0.5 * lv)
    z = _prelu(_dot(lat, dwlat_ref[...]) + ge_mm(dwge_ref)
               + dbeff_ref[...], d0_ref)
    z = _prelu(_dot(z, dw1_ref[...]) + db1_ref[...], d1_ref)
    z = _prelu(_dot(z, dw2_ref[...]) + db2_ref[...], d2_ref)
    y_ref[...] = jnp.clip(_dot(z, dw3_ref[...]) + db3_ref[...], -10.0, 10.0)


def _vae_mlp(labp, g, eps, ws):
    return pl.pallas_call(
        _vae_body,
        out_shape=[
            jax.ShapeDtypeStruct((B, 128), _F32),
            jax.ShapeDtypeStruct((B, LAT), _F32),
            jax.ShapeDtypeStruct((B, LAT), _F32),
        ],
    )(labp, g, eps, *ws)


def kernel(r1_x, r1_e, r1_src, r1_dst, r1_gid, r2_x, r2_e, r2_src, r2_dst,
           r2_gid, pm_x, pm_e, pm_src, pm_dst, pm_gid, labels,
           pos_neg_sample, params):
    p = params
    f32 = _F32

    x = jnp.concatenate([r1_x, r2_x, pm_x], axis=0)
    src = jnp.concatenate(
        [r1_src, r2_src + N, pm_src + 2 * N]).astype(jnp.int32).reshape(NW, NCH, 128)
    dst = jnp.concatenate(
        [r1_dst, r2_dst + N, pm_dst + 2 * N]).astype(jnp.int32).reshape(16, NCS, 128)
    gidc = jnp.concatenate(
        [r1_gid, r2_gid + B, pm_gid + 2 * B]).astype(f32).reshape(NT, 1)

    w2 = p['edge_W'].reshape(DE * H, H)
    evt = jnp.concatenate([r1_e, r2_e, pm_e], axis=0).T
    bmat = p['edge_b'].reshape(H, H)
    cb = p['conv_b'].reshape(1, H)
    gws = ([p['gru_Wih'][k * H:(k + 1) * H].T for k in range(3)]
           + [p['gru_Whh'][k * H:(k + 1) * H].T for k in range(3)])
    gbs = ([p['gru_bih'][k * H:(k + 1) * H].reshape(1, H) for k in range(3)]
           + [p['gru_bhh'][k * H:(k + 1) * H].reshape(1, H) for k in range(3)])

    f = _proj(x, p['proj_W'], p['proj_b'].reshape(1, H))
    for _ in range(3):
        hs = _sc_gather(f, src)
        m = _msg(hs, evt, w2, bmat)
        agg = _sc_scatter(m, dst)
        f = _gru(agg, f, cb, gws, gbs)

    feat = f

    lwq = p['lstm_Wih'][:, :2 * H].T
    lwr = p['lstm_Wih'][:, 2 * H:].T
    lwh = p['lstm_Whh'].T
    lbi = p['lstm_bih'].reshape(1, 8 * H)
    lbh = p['lstm_bhh'].reshape(1, 8 * H)
    q = jnp.zeros((BT, 2 * H), f32)
    run = jnp.zeros((BT, 2 * H), f32)
    den = jnp.ones((BT, 1), f32)
    hl = jnp.zeros((BT, 2 * H), f32)
    cl = jnp.zeros((BT, 2 * H), f32)
    for _ in range(3):
        hl, cl = _lstm(q, run, den, hl, cl, lwq, lwr, lwh, lbi, lbh)
        q = hl
        _, den, run = _ro(feat, gidc, q)

    g = _sp(q, run, den, p['sp_W'][:2 * H], p['sp_W'][2 * H:],
            p['sp_b'].reshape(1, RO), p['sp_a'].reshape(1, 1))

    posf = jnp.asarray(pos_neg_sample).astype(f32)
    labp = jnp.pad(labels, ((0, 0), (0, 128 - NC)))
    eps = jax.random.normal(jax.random.key(42), (B, LAT), f32)
    gei = NC + NG * RO
    ws = [
        jnp.pad(p['enc_W0'][:NC], ((0, 128 - NC), (0, 0))),
        p['enc_W0'][NC:gei],
        (p['enc_b0'] + posf * p['enc_W0'][gei]).reshape(1, PH),
        p['enc_a0'].reshape(1, 1),
        p['enc_W1'], p['enc_b1'].reshape(1, PH), p['enc_a1'].reshape(1, 1),
        p['enc_W2'], p['enc_b2'].reshape(1, PH), p['enc_a2'].reshape(1, 1),
        p['enc_W3'], p['enc_b3'].reshape(1, 2 * LAT),
        p['dec_W0'][:LAT],
        p['dec_W0'][LAT:LAT + NG * RO],
        (p['dec_b0'] + posf * p['dec_W0'][LAT + NG * RO]).reshape(1, PH),
        p['dec_a0'].reshape(1, 1),
        p['dec_W1'], p['dec_b1'].reshape(1, PH), p['dec_a1'].reshape(1, 1),
        p['dec_W2'], p['dec_b2'].reshape(1, PH), p['dec_a2'].reshape(1, 1),
        jnp.pad(p['dec_W3'], ((0, 0), (0, 128 - NC))),
        jnp.pad(p['dec_b3'], (0, 128 - NC)).reshape(1, 128),
    ]
    y, mu, log_var = _vae_mlp(labp, g, eps, ws)
    y = y[:, :NC]
    return (y, mu, log_var)
